# Initial kernel scaffold; baseline (speedup 1.0000x reference)
import jax
import jax.numpy as jnp
from jax.experimental import pallas as pl


def kernel(feat, shuf_feat, edge_index, W1, b1, Wlin, blin, Wbil, bbil, prelu_w):
    raise NotImplementedError("write your pallas kernel here")


# speedup vs baseline: 3.4518x; 3.4518x over previous
import functools

import jax
import jax.numpy as jnp
from jax import lax
from jax.experimental import pallas as pl
from jax.experimental.pallas import tpu as pltpu
from jax.experimental.pallas import tpu_sc as plsc

_N = 10000
_E = 320000
_D = 128
_K = 10
_ALPHA = 0.1

_NS = 16
_L = 16

_NPAD = 10240
_RPT = _NPAD // _NS
_CH = 64
_NRCH = _RPT // _CH
_ECH = 313
_EPT = _ECH * _CH
_EPAD = _NS * _EPT


def _deg_body(src_ref, dst_ref, dego_ref, degi_ref, idx_v, ones_v, stage_v,
              deg_sh):
    cid = lax.axis_index("c")
    sid = lax.axis_index("s")

    def fill1(i, _):
        ones_v[pl.ds(i * _L, _L)] = jnp.ones((_L,), jnp.float32)
        return 0
    lax.fori_loop(0, _CH // _L, fill1, 0)

    def fill0(i, _):
        stage_v[pl.ds(i * _L, _L)] = jnp.zeros((_L,), jnp.float32)
        return 0
    lax.fori_loop(0, _RPT // _L, fill0, 0)
    pltpu.sync_copy(stage_v, deg_sh.at[pl.ds(sid * _RPT, _RPT)])
    plsc.subcore_barrier()

    def chunk(ch, _):
        base = sid * _EPT + ch * _CH

        @pl.when(cid == 0)
        def _():
            pltpu.sync_copy(src_ref.at[pl.ds(base, _CH)], idx_v)

        @pl.when(cid == 1)
        def _():
            pltpu.sync_copy(dst_ref.at[pl.ds(base, _CH)], idx_v)

        pltpu.sync_copy(ones_v, deg_sh.at[idx_v], add=True)
        return 0
    lax.fori_loop(0, _ECH, chunk, 0)
    plsc.subcore_barrier()

    pltpu.sync_copy(deg_sh.at[pl.ds(sid * _RPT, _RPT)], stage_v)

    @pl.when(cid == 0)
    def _():
        pltpu.sync_copy(stage_v, dego_ref.at[pl.ds(sid * _RPT, _RPT)])

    @pl.when(cid == 1)
    def _():
        pltpu.sync_copy(stage_v, degi_ref.at[pl.ds(sid * _RPT, _RPT)])


_PBLK = _NPAD // 8


def _prep_body(degT_ref, x_ref, cn_ref, nd_ref, g0_ref):
    dg = degT_ref[...]
    ns = lax.rsqrt(jnp.maximum(dg[:, 0:1], 1.0))
    ndv = lax.rsqrt(jnp.maximum(dg[:, 1:2], 1.0))
    cn_ref[...] = jnp.broadcast_to((1.0 - _ALPHA) * ns * ndv, (_PBLK, _D))
    nd_ref[...] = jnp.broadcast_to(ndv, (_PBLK, _D))
    g0_ref[...] = ns * x_ref[...]


def _loop_body(src_ref, dst_ref, g0_ref, x_ref, cn_ref, nd_ref,
               gw_ref, hg_ref, hk_ref,
               idx_s, idx_d, buf_a, buf_b, buf_c, buf_d, agg_sh, sem):
    cid = lax.axis_index("c")
    sid = lax.axis_index("s")
    row0 = sid * _RPT
    coff = cid * _NPAD

    def zrow(r, _):
        for c in range(_D // _L):
            buf_d[r, pl.ds(c * _L, _L)] = jnp.zeros((_L,), jnp.float32)
        return 0
    lax.fori_loop(0, _CH, zrow, 0)

    def init_chunk(rb, _):
        r = row0 + rb * _CH
        pltpu.sync_copy(g0_ref.at[pl.ds(coff + r, _CH)], buf_a)
        pltpu.sync_copy(buf_a, gw_ref.at[pl.ds(coff + r, _CH)])
        pltpu.sync_copy(buf_d, agg_sh.at[pl.ds(r, _CH)])
        return 0
    lax.fori_loop(0, _NRCH, init_chunk, 0)
    plsc.subcore_barrier()

    def edge_phase():
        def chunk(ch, _):
            base = sid * _EPT + ch * _CH
            pltpu.sync_copy(src_ref.at[pl.ds(base, _CH)], idx_s)
            pltpu.sync_copy(dst_ref.at[pl.ds(base, _CH)], idx_d)
            for c in range(_CH // _L):
                idx_s[pl.ds(c * _L, _L)] = idx_s[pl.ds(c * _L, _L)] + coff
            pltpu.async_copy(gw_ref.at[idx_s], buf_a, sem).wait()
            pltpu.sync_copy(buf_a, agg_sh.at[idx_d], add=True)
            return 0
        lax.fori_loop(0, _ECH, chunk, 0)

    def ew(fn):
        def row(r, _):
            for c in range(_D // _L):
                fn(r, c * _L)
            return 0
        lax.fori_loop(0, _CH, row, 0)

    for t in range(_K):
        edge_phase()
        plsc.subcore_barrier()

        last = (t == _K - 1)

        def dense_chunk(rb, _, t=t, last=last):
            r = row0 + rb * _CH
            rg = coff + r
            pltpu.sync_copy(agg_sh.at[pl.ds(r, _CH)], buf_a)
            if t == 0:
                pltpu.sync_copy(nd_ref.at[pl.ds(r, _CH)], buf_b)

                def body_hg(rr, cs):
                    buf_b[rr, pl.ds(cs, _L)] = (
                        buf_b[rr, pl.ds(cs, _L)] * buf_a[rr, pl.ds(cs, _L)])
                ew(body_hg)
                pltpu.sync_copy(buf_b, hg_ref.at[pl.ds(rg, _CH)])
            if not last:
                pltpu.sync_copy(cn_ref.at[pl.ds(r, _CH)], buf_b)
                pltpu.sync_copy(g0_ref.at[pl.ds(rg, _CH)], buf_c)

                def body_g(rr, cs):
                    buf_a[rr, pl.ds(cs, _L)] = (
                        buf_b[rr, pl.ds(cs, _L)] * buf_a[rr, pl.ds(cs, _L)]
                        + _ALPHA * buf_c[rr, pl.ds(cs, _L)])
                ew(body_g)
                pltpu.sync_copy(buf_a, gw_ref.at[pl.ds(rg, _CH)])
                pltpu.sync_copy(buf_d, agg_sh.at[pl.ds(r, _CH)])
            else:
                pltpu.sync_copy(nd_ref.at[pl.ds(r, _CH)], buf_b)
                pltpu.sync_copy(x_ref.at[pl.ds(rg, _CH)], buf_c)

                def body_hk(rr, cs):
                    buf_a[rr, pl.ds(cs, _L)] = (
                        (1.0 - _ALPHA)
                        * buf_b[rr, pl.ds(cs, _L)] * buf_a[rr, pl.ds(cs, _L)]
                        + _ALPHA * buf_c[rr, pl.ds(cs, _L)])
                ew(body_hk)
                pltpu.sync_copy(buf_a, hk_ref.at[pl.ds(rg, _CH)])
            return 0
        lax.fori_loop(0, _NRCH, dense_chunk, 0)
        plsc.subcore_barrier()


_NBLK = 2000


def _prelu(x, w):
    return jnp.where(x > 0, x, w * x)


def _sums_body(hgf_ref, hkf_ref, w1_ref, b1_ref, wl_ref, bl_ref, pw_ref,
               sums_ref):
    i = pl.program_id(0)
    w = pw_ref[0, 0]
    h1 = _prelu(jnp.dot(hgf_ref[...], w1_ref[...],
                        preferred_element_type=jnp.float32) + b1_ref[...], w)
    h2 = _prelu(jnp.dot(hkf_ref[...], wl_ref[...],
                        preferred_element_type=jnp.float32) + bl_ref[...], w)

    @pl.when(i == 0)
    def _():
        sums_ref[...] = jnp.zeros((8, _D), jnp.float32)

    sums_ref[0:1, :] = sums_ref[0:1, :] + jnp.sum(h1, axis=0, keepdims=True)
    sums_ref[1:2, :] = sums_ref[1:2, :] + jnp.sum(h2, axis=0, keepdims=True)


def _scores_body(sums_ref, hgf_ref, hkf_ref, hgs_ref, hks_ref,
                 w1_ref, b1_ref, wl_ref, bl_ref, wb_ref, bb_ref, pw_ref,
                 out_ref):
    w = pw_ref[0, 0]
    bb = bb_ref[0, 0]
    c1 = jax.nn.sigmoid(sums_ref[0:1, :] * (1.0 / _N))
    c2 = jax.nn.sigmoid(sums_ref[1:2, :] * (1.0 / _N))
    q1 = lax.dot_general(wb_ref[...], c1, (((1,), (1,)), ((), ())),
                         preferred_element_type=jnp.float32)
    q2 = lax.dot_general(wb_ref[...], c2, (((1,), (1,)), ((), ())),
                         preferred_element_type=jnp.float32)
    h1 = _prelu(jnp.dot(hgf_ref[...], w1_ref[...],
                        preferred_element_type=jnp.float32) + b1_ref[...], w)
    h2 = _prelu(jnp.dot(hkf_ref[...], wl_ref[...],
                        preferred_element_type=jnp.float32) + bl_ref[...], w)
    h3 = _prelu(jnp.dot(hgs_ref[...], w1_ref[...],
                        preferred_element_type=jnp.float32) + b1_ref[...], w)
    h4 = _prelu(jnp.dot(hks_ref[...], wl_ref[...],
                        preferred_element_type=jnp.float32) + bl_ref[...], w)
    out_ref[:, 0:1] = jnp.dot(h2, q1, preferred_element_type=jnp.float32) + bb
    out_ref[:, 1:2] = jnp.dot(h1, q2, preferred_element_type=jnp.float32) + bb
    out_ref[:, 2:3] = jnp.dot(h4, q1, preferred_element_type=jnp.float32) + bb
    out_ref[:, 3:4] = jnp.dot(h3, q2, preferred_element_type=jnp.float32) + bb


@functools.cache
def _build_calls():
    sc_mesh = plsc.VectorSubcoreMesh(core_axis_name="c", subcore_axis_name="s")
    deg_call = pl.kernel(
        _deg_body,
        out_type=(
            jax.ShapeDtypeStruct((_NPAD,), jnp.float32),
            jax.ShapeDtypeStruct((_NPAD,), jnp.float32),
        ),
        mesh=sc_mesh,
        scratch_types=[
            pltpu.VMEM((_CH,), jnp.int32),
            pltpu.VMEM((_CH,), jnp.float32),
            pltpu.VMEM((_RPT,), jnp.float32),
            pltpu.VMEM_SHARED((_NPAD,), jnp.float32),
        ],
    )
    prep_call = pl.pallas_call(
        _prep_body,
        grid=(2, 8),
        in_specs=[
            pl.BlockSpec((_PBLK, 2), lambda c, b: (b, 0)),
            pl.BlockSpec((_PBLK, _D), lambda c, b: (c * 8 + b, 0)),
        ],
        out_specs=[
            pl.BlockSpec((_PBLK, _D), lambda c, b: (b, 0)),
            pl.BlockSpec((_PBLK, _D), lambda c, b: (b, 0)),
            pl.BlockSpec((_PBLK, _D), lambda c, b: (c * 8 + b, 0)),
        ],
        out_shape=[
            jax.ShapeDtypeStruct((_NPAD, _D), jnp.float32),
            jax.ShapeDtypeStruct((_NPAD, _D), jnp.float32),
            jax.ShapeDtypeStruct((2 * _NPAD, _D), jnp.float32),
        ],
    )
    loop_call = pl.kernel(
        _loop_body,
        out_type=(
            jax.ShapeDtypeStruct((2 * _NPAD, _D), jnp.float32),
            jax.ShapeDtypeStruct((2 * _NPAD, _D), jnp.float32),
            jax.ShapeDtypeStruct((2 * _NPAD, _D), jnp.float32),
        ),
        mesh=sc_mesh,
        scratch_types=[
            pltpu.VMEM((_CH,), jnp.int32),
            pltpu.VMEM((_CH,), jnp.int32),
            pltpu.VMEM((_CH, _D), jnp.float32),
            pltpu.VMEM((_CH, _D), jnp.float32),
            pltpu.VMEM((_CH, _D), jnp.float32),
            pltpu.VMEM((_CH, _D), jnp.float32),
            pltpu.VMEM_SHARED((_NPAD, _D), jnp.float32),
            pltpu.SemaphoreType.DMA,
        ],
    )
    sums_call = pl.pallas_call(
        _sums_body,
        grid=(_N // _NBLK,),
        in_specs=[
            pl.BlockSpec((_NBLK, _D), lambda b: (b, 0)),
            pl.BlockSpec((_NBLK, _D), lambda b: (b, 0)),
            pl.BlockSpec((_D, _D), lambda b: (0, 0)),
            pl.BlockSpec((1, _D), lambda b: (0, 0)),
            pl.BlockSpec((_D, _D), lambda b: (0, 0)),
            pl.BlockSpec((1, _D), lambda b: (0, 0)),
            pl.BlockSpec((1, 1), lambda b: (0, 0)),
        ],
        out_specs=pl.BlockSpec((8, _D), lambda b: (0, 0)),
        out_shape=jax.ShapeDtypeStruct((8, _D), jnp.float32),
    )
    scores_call = pl.pallas_call(
        _scores_body,
        grid=(_N // _NBLK,),
        in_specs=[
            pl.BlockSpec((8, _D), lambda b: (0, 0)),
            pl.BlockSpec((_NBLK, _D), lambda b: (b, 0)),
            pl.BlockSpec((_NBLK, _D), lambda b: (b, 0)),
            pl.BlockSpec((_NBLK, _D), lambda b: (b, 0)),
            pl.BlockSpec((_NBLK, _D), lambda b: (b, 0)),
            pl.BlockSpec((_D, _D), lambda b: (0, 0)),
            pl.BlockSpec((1, _D), lambda b: (0, 0)),
            pl.BlockSpec((_D, _D), lambda b: (0, 0)),
            pl.BlockSpec((1, _D), lambda b: (0, 0)),
            pl.BlockSpec((_D, _D), lambda b: (0, 0)),
            pl.BlockSpec((1, 1), lambda b: (0, 0)),
            pl.BlockSpec((1, 1), lambda b: (0, 0)),
        ],
        out_specs=pl.BlockSpec((_NBLK, 4), lambda b: (b, 0)),
        out_shape=jax.ShapeDtypeStruct((_N, 4), jnp.float32),
    )
    return deg_call, prep_call, loop_call, sums_call, scores_call


def kernel(feat, shuf_feat, edge_index, W1, b1, Wlin, blin, Wbil, bbil,
           prelu_w):
    deg_call, prep_call, loop_call, sums_call, scores_call = _build_calls()

    src = edge_index[0].astype(jnp.int32)
    dst = edge_index[1].astype(jnp.int32)
    pad_ids = _N + (jnp.arange(_EPAD - _E, dtype=jnp.int32) % (_NPAD - _N))
    srcp = jnp.concatenate([src, pad_ids])
    dstp = jnp.concatenate([dst, pad_ids])

    xf = jnp.pad(feat, ((0, _NPAD - _N), (0, 0)))
    xs = jnp.pad(shuf_feat, ((0, _NPAD - _N), (0, 0)))
    xcat = jnp.concatenate([xf, xs], axis=0)

    dego, degi = deg_call(srcp, dstp)
    degT = jnp.stack([dego, degi], axis=1)
    cn, nd, g0cat = prep_call(degT, xcat)
    _, hgcat, hkcat = loop_call(srcp, dstp, g0cat, xcat, cn, nd)

    hgf = hgcat[:_N]
    hgs = hgcat[_NPAD:_NPAD + _N]
    hkf = hkcat[:_N]
    hks = hkcat[_NPAD:_NPAD + _N]

    b1r = b1.reshape(1, _D)
    blr = blin.reshape(1, _D)
    pwr = prelu_w.reshape(1, 1)
    bbr = bbil.reshape(1, 1)

    sums = sums_call(hgf, hkf, W1, b1r, Wlin, blr, pwr)
    scores = scores_call(sums, hgf, hkf, hgs, hks, W1, b1r, Wlin, blr,
                         Wbil, bbr, pwr)
    return scores.T.reshape(4 * _N)


# speedup vs baseline: 5.7193x; 1.6569x over previous
import functools

import jax
import jax.numpy as jnp
from jax import lax
from jax.experimental import pallas as pl
from jax.experimental.pallas import tpu as pltpu
from jax.experimental.pallas import tpu_sc as plsc

_N = 10000
_E = 320000
_D = 128
_K = 10
_ALPHA = 0.1

_NS = 16
_L = 16

_NPAD = 10240
_RPT = _NPAD // _NS
_CH = 64
_NRCH = _RPT // _CH
_U = 4
_NBODY = 79
_ECH = _U * _NBODY
_EPT = _ECH * _CH
_EPAD = _NS * _EPT


def _deg_body(src_ref, dst_ref, dego_ref, degi_ref, idx_v, ones_v, stage_v,
              deg_sh):
    cid = lax.axis_index("c")
    sid = lax.axis_index("s")

    def fill1(i, _):
        ones_v[pl.ds(i * _L, _L)] = jnp.ones((_L,), jnp.float32)
        return 0
    lax.fori_loop(0, _CH // _L, fill1, 0)

    def fill0(i, _):
        stage_v[pl.ds(i * _L, _L)] = jnp.zeros((_L,), jnp.float32)
        return 0
    lax.fori_loop(0, _RPT // _L, fill0, 0)
    pltpu.sync_copy(stage_v, deg_sh.at[pl.ds(sid * _RPT, _RPT)])
    plsc.subcore_barrier()

    def chunk(ch, _):
        row = sid * _ECH + ch

        @pl.when(cid == 0)
        def _():
            pltpu.sync_copy(src_ref.at[row], idx_v)

        @pl.when(cid == 1)
        def _():
            pltpu.sync_copy(dst_ref.at[row], idx_v)

        pltpu.sync_copy(ones_v, deg_sh.at[idx_v], add=True)
        return 0
    lax.fori_loop(0, _ECH, chunk, 0)
    plsc.subcore_barrier()

    pltpu.sync_copy(deg_sh.at[pl.ds(sid * _RPT, _RPT)], stage_v)

    @pl.when(cid == 0)
    def _():
        pltpu.sync_copy(stage_v, dego_ref.at[pl.ds(sid * _RPT, _RPT)])

    @pl.when(cid == 1)
    def _():
        pltpu.sync_copy(stage_v, degi_ref.at[pl.ds(sid * _RPT, _RPT)])


_PBLK = _NPAD // 8


def _prep_body(degT_ref, x_ref, cn_ref, nd_ref, g0_ref):
    dg = degT_ref[...]
    ns = lax.rsqrt(jnp.maximum(dg[:, 0:1], 1.0))
    ndv = lax.rsqrt(jnp.maximum(dg[:, 1:2], 1.0))
    cn_ref[...] = jnp.broadcast_to((1.0 - _ALPHA) * ns * ndv, (_PBLK, _D))
    nd_ref[...] = jnp.broadcast_to(ndv, (_PBLK, _D))
    g0_ref[...] = ns * x_ref[...]


def _loop_body(src_ref, dst_ref, g0_ref, x_ref, cn_ref, nd_ref,
               gw_ref, hg_ref, hk_ref,
               idx_sb, idx_db, buf_a, buf_b, buf_c, buf_d, agg_sh,
               sga, sgb, ssa, ssb):
    cid = lax.axis_index("c")
    sid = lax.axis_index("s")
    row0 = sid * _RPT
    coff = cid * _NPAD

    def zrow(r, _):
        for c in range(_D // _L):
            buf_d[r, pl.ds(c * _L, _L)] = jnp.zeros((_L,), jnp.float32)
        return 0
    lax.fori_loop(0, _CH, zrow, 0)

    def init_chunk(rb, _):
        r = row0 + rb * _CH
        pltpu.sync_copy(g0_ref.at[pl.ds(coff + r, _CH)], buf_a)
        pltpu.sync_copy(buf_a, gw_ref.at[pl.ds(coff + r, _CH)])
        pltpu.sync_copy(buf_d, agg_sh.at[pl.ds(r, _CH)])
        return 0
    lax.fori_loop(0, _NRCH, init_chunk, 0)
    plsc.subcore_barrier()

    def edge_phase():
        def body(j, _):
            brow = sid * _ECH + j * _U
            pltpu.sync_copy(src_ref.at[pl.ds(brow, _U)], idx_sb)
            pltpu.sync_copy(dst_ref.at[pl.ds(brow, _U)], idx_db)
            for k in range(_U):
                for c in range(_CH // _L):
                    idx_sb[k, pl.ds(c * _L, _L)] = (
                        idx_sb[k, pl.ds(c * _L, _L)] + coff)
            dg0 = pltpu.async_copy(gw_ref.at[idx_sb.at[0]], buf_a, sga)
            dg1 = pltpu.async_copy(gw_ref.at[idx_sb.at[1]], buf_b, sgb)
            dg0.wait()
            ds0 = pltpu.async_copy(buf_a, agg_sh.at[idx_db.at[0]], ssa,
                                   add=True)
            dg1.wait()
            ds1 = pltpu.async_copy(buf_b, agg_sh.at[idx_db.at[1]], ssb,
                                   add=True)
            ds0.wait()
            dg2 = pltpu.async_copy(gw_ref.at[idx_sb.at[2]], buf_a, sga)
            ds1.wait()
            dg3 = pltpu.async_copy(gw_ref.at[idx_sb.at[3]], buf_b, sgb)
            dg2.wait()
            ds2 = pltpu.async_copy(buf_a, agg_sh.at[idx_db.at[2]], ssa,
                                   add=True)
            dg3.wait()
            ds3 = pltpu.async_copy(buf_b, agg_sh.at[idx_db.at[3]], ssb,
                                   add=True)
            ds2.wait()
            ds3.wait()
            return 0
        lax.fori_loop(0, _NBODY, body, 0)

    def ew(fn):
        def row(r, _):
            for c in range(_D // _L):
                fn(r, c * _L)
            return 0
        lax.fori_loop(0, _CH, row, 0)

    for t in range(_K):
        edge_phase()
        plsc.subcore_barrier()

        last = (t == _K - 1)

        def dense_chunk(rb, _, t=t, last=last):
            r = row0 + rb * _CH
            rg = coff + r
            pltpu.sync_copy(agg_sh.at[pl.ds(r, _CH)], buf_a)
            if t == 0:
                pltpu.sync_copy(nd_ref.at[pl.ds(r, _CH)], buf_b)

                def body_hg(rr, cs):
                    buf_b[rr, pl.ds(cs, _L)] = (
                        buf_b[rr, pl.ds(cs, _L)] * buf_a[rr, pl.ds(cs, _L)])
                ew(body_hg)
                pltpu.sync_copy(buf_b, hg_ref.at[pl.ds(rg, _CH)])
            if not last:
                pltpu.sync_copy(cn_ref.at[pl.ds(r, _CH)], buf_b)
                pltpu.sync_copy(g0_ref.at[pl.ds(rg, _CH)], buf_c)

                def body_g(rr, cs):
                    buf_a[rr, pl.ds(cs, _L)] = (
                        buf_b[rr, pl.ds(cs, _L)] * buf_a[rr, pl.ds(cs, _L)]
                        + _ALPHA * buf_c[rr, pl.ds(cs, _L)])
                ew(body_g)
                pltpu.sync_copy(buf_a, gw_ref.at[pl.ds(rg, _CH)])
                pltpu.sync_copy(buf_d, agg_sh.at[pl.ds(r, _CH)])
            else:
                pltpu.sync_copy(nd_ref.at[pl.ds(r, _CH)], buf_b)
                pltpu.sync_copy(x_ref.at[pl.ds(rg, _CH)], buf_c)

                def body_hk(rr, cs):
                    buf_a[rr, pl.ds(cs, _L)] = (
                        (1.0 - _ALPHA)
                        * buf_b[rr, pl.ds(cs, _L)] * buf_a[rr, pl.ds(cs, _L)]
                        + _ALPHA * buf_c[rr, pl.ds(cs, _L)])
                ew(body_hk)
                pltpu.sync_copy(buf_a, hk_ref.at[pl.ds(rg, _CH)])
            return 0
        lax.fori_loop(0, _NRCH, dense_chunk, 0)
        plsc.subcore_barrier()


_NBLK = 2000


def _prelu(x, w):
    return jnp.where(x > 0, x, w * x)


def _sums_body(hgf_ref, hkf_ref, w1_ref, b1_ref, wl_ref, bl_ref, pw_ref,
               sums_ref):
    i = pl.program_id(0)
    w = pw_ref[0, 0]
    h1 = _prelu(jnp.dot(hgf_ref[...], w1_ref[...],
                        preferred_element_type=jnp.float32) + b1_ref[...], w)
    h2 = _prelu(jnp.dot(hkf_ref[...], wl_ref[...],
                        preferred_element_type=jnp.float32) + bl_ref[...], w)

    @pl.when(i == 0)
    def _():
        sums_ref[...] = jnp.zeros((8, _D), jnp.float32)

    sums_ref[0:1, :] = sums_ref[0:1, :] + jnp.sum(h1, axis=0, keepdims=True)
    sums_ref[1:2, :] = sums_ref[1:2, :] + jnp.sum(h2, axis=0, keepdims=True)


def _scores_body(sums_ref, hgf_ref, hkf_ref, hgs_ref, hks_ref,
                 w1_ref, b1_ref, wl_ref, bl_ref, wb_ref, bb_ref, pw_ref,
                 out_ref):
    w = pw_ref[0, 0]
    bb = bb_ref[0, 0]
    c1 = jax.nn.sigmoid(sums_ref[0:1, :] * (1.0 / _N))
    c2 = jax.nn.sigmoid(sums_ref[1:2, :] * (1.0 / _N))
    q1 = lax.dot_general(wb_ref[...], c1, (((1,), (1,)), ((), ())),
                         preferred_element_type=jnp.float32)
    q2 = lax.dot_general(wb_ref[...], c2, (((1,), (1,)), ((), ())),
                         preferred_element_type=jnp.float32)
    h1 = _prelu(jnp.dot(hgf_ref[...], w1_ref[...],
                        preferred_element_type=jnp.float32) + b1_ref[...], w)
    h2 = _prelu(jnp.dot(hkf_ref[...], wl_ref[...],
                        preferred_element_type=jnp.float32) + bl_ref[...], w)
    h3 = _prelu(jnp.dot(hgs_ref[...], w1_ref[...],
                        preferred_element_type=jnp.float32) + b1_ref[...], w)
    h4 = _prelu(jnp.dot(hks_ref[...], wl_ref[...],
                        preferred_element_type=jnp.float32) + bl_ref[...], w)
    out_ref[:, 0:1] = jnp.dot(h2, q1, preferred_element_type=jnp.float32) + bb
    out_ref[:, 1:2] = jnp.dot(h1, q2, preferred_element_type=jnp.float32) + bb
    out_ref[:, 2:3] = jnp.dot(h4, q1, preferred_element_type=jnp.float32) + bb
    out_ref[:, 3:4] = jnp.dot(h3, q2, preferred_element_type=jnp.float32) + bb


@functools.cache
def _build_calls():
    sc_mesh = plsc.VectorSubcoreMesh(core_axis_name="c", subcore_axis_name="s")
    deg_call = pl.kernel(
        _deg_body,
        out_type=(
            jax.ShapeDtypeStruct((_NPAD,), jnp.float32),
            jax.ShapeDtypeStruct((_NPAD,), jnp.float32),
        ),
        mesh=sc_mesh,
        scratch_types=[
            pltpu.VMEM((_CH,), jnp.int32),
            pltpu.VMEM((_CH,), jnp.float32),
            pltpu.VMEM((_RPT,), jnp.float32),
            pltpu.VMEM_SHARED((_NPAD,), jnp.float32),
        ],
    )
    prep_call = pl.pallas_call(
        _prep_body,
        grid=(2, 8),
        in_specs=[
            pl.BlockSpec((_PBLK, 2), lambda c, b: (b, 0)),
            pl.BlockSpec((_PBLK, _D), lambda c, b: (c * 8 + b, 0)),
        ],
        out_specs=[
            pl.BlockSpec((_PBLK, _D), lambda c, b: (b, 0)),
            pl.BlockSpec((_PBLK, _D), lambda c, b: (b, 0)),
            pl.BlockSpec((_PBLK, _D), lambda c, b: (c * 8 + b, 0)),
        ],
        out_shape=[
            jax.ShapeDtypeStruct((_NPAD, _D), jnp.float32),
            jax.ShapeDtypeStruct((_NPAD, _D), jnp.float32),
            jax.ShapeDtypeStruct((2 * _NPAD, _D), jnp.float32),
        ],
    )
    loop_call = pl.kernel(
        _loop_body,
        out_type=(
            jax.ShapeDtypeStruct((2 * _NPAD, _D), jnp.float32),
            jax.ShapeDtypeStruct((2 * _NPAD, _D), jnp.float32),
            jax.ShapeDtypeStruct((2 * _NPAD, _D), jnp.float32),
        ),
        mesh=sc_mesh,
        scratch_types=[
            pltpu.VMEM((_U, _CH), jnp.int32),
            pltpu.VMEM((_U, _CH), jnp.int32),
            pltpu.VMEM((_CH, _D), jnp.float32),
            pltpu.VMEM((_CH, _D), jnp.float32),
            pltpu.VMEM((_CH, _D), jnp.float32),
            pltpu.VMEM((_CH, _D), jnp.float32),
            pltpu.VMEM_SHARED((_NPAD, _D), jnp.float32),
            pltpu.SemaphoreType.DMA,
            pltpu.SemaphoreType.DMA,
            pltpu.SemaphoreType.DMA,
            pltpu.SemaphoreType.DMA,
        ],
    )
    sums_call = pl.pallas_call(
        _sums_body,
        grid=(_N // _NBLK,),
        in_specs=[
            pl.BlockSpec((_NBLK, _D), lambda b: (b, 0)),
            pl.BlockSpec((_NBLK, _D), lambda b: (b, 0)),
            pl.BlockSpec((_D, _D), lambda b: (0, 0)),
            pl.BlockSpec((1, _D), lambda b: (0, 0)),
            pl.BlockSpec((_D, _D), lambda b: (0, 0)),
            pl.BlockSpec((1, _D), lambda b: (0, 0)),
            pl.BlockSpec((1, 1), lambda b: (0, 0)),
        ],
        out_specs=pl.BlockSpec((8, _D), lambda b: (0, 0)),
        out_shape=jax.ShapeDtypeStruct((8, _D), jnp.float32),
    )
    scores_call = pl.pallas_call(
        _scores_body,
        grid=(_N // _NBLK,),
        in_specs=[
            pl.BlockSpec((8, _D), lambda b: (0, 0)),
            pl.BlockSpec((_NBLK, _D), lambda b: (b, 0)),
            pl.BlockSpec((_NBLK, _D), lambda b: (b, 0)),
            pl.BlockSpec((_NBLK, _D), lambda b: (b, 0)),
            pl.BlockSpec((_NBLK, _D), lambda b: (b, 0)),
            pl.BlockSpec((_D, _D), lambda b: (0, 0)),
            pl.BlockSpec((1, _D), lambda b: (0, 0)),
            pl.BlockSpec((_D, _D), lambda b: (0, 0)),
            pl.BlockSpec((1, _D), lambda b: (0, 0)),
            pl.BlockSpec((_D, _D), lambda b: (0, 0)),
            pl.BlockSpec((1, 1), lambda b: (0, 0)),
            pl.BlockSpec((1, 1), lambda b: (0, 0)),
        ],
        out_specs=pl.BlockSpec((_NBLK, 4), lambda b: (b, 0)),
        out_shape=jax.ShapeDtypeStruct((_N, 4), jnp.float32),
    )
    return deg_call, prep_call, loop_call, sums_call, scores_call


def kernel(feat, shuf_feat, edge_index, W1, b1, Wlin, blin, Wbil, bbil,
           prelu_w):
    deg_call, prep_call, loop_call, sums_call, scores_call = _build_calls()

    src = edge_index[0].astype(jnp.int32)
    dst = edge_index[1].astype(jnp.int32)
    pad_ids = _N + (jnp.arange(_EPAD - _E, dtype=jnp.int32) % (_NPAD - _N))
    srcp = jnp.concatenate([src, pad_ids]).reshape(_EPAD // _CH, _CH)
    dstp = jnp.concatenate([dst, pad_ids]).reshape(_EPAD // _CH, _CH)

    xf = jnp.pad(feat, ((0, _NPAD - _N), (0, 0)))
    xs = jnp.pad(shuf_feat, ((0, _NPAD - _N), (0, 0)))
    xcat = jnp.concatenate([xf, xs], axis=0)

    dego, degi = deg_call(srcp, dstp)
    degT = jnp.stack([dego, degi], axis=1)
    cn, nd, g0cat = prep_call(degT, xcat)
    _, hgcat, hkcat = loop_call(srcp, dstp, g0cat, xcat, cn, nd)

    hgf = hgcat[:_N]
    hgs = hgcat[_NPAD:_NPAD + _N]
    hkf = hkcat[:_N]
    hks = hkcat[_NPAD:_NPAD + _N]

    b1r = b1.reshape(1, _D)
    blr = blin.reshape(1, _D)
    pwr = prelu_w.reshape(1, 1)
    bbr = bbil.reshape(1, 1)

    sums = sums_call(hgf, hkf, W1, b1r, Wlin, blr, pwr)
    scores = scores_call(sums, hgf, hkf, hgs, hks, W1, b1r, Wlin, blr,
                         Wbil, bbr, pwr)
    return scores.T.reshape(4 * _N)


# speedup vs baseline: 9.3959x; 1.6428x over previous
import functools

import jax
import jax.numpy as jnp
from jax import lax
from jax.experimental import pallas as pl
from jax.experimental.pallas import tpu as pltpu
from jax.experimental.pallas import tpu_sc as plsc

_N = 10000
_E = 320000
_D = 128
_K = 10
_ALPHA = 0.1

_NS = 16
_L = 16

_NPAD = 10240
_RPT = _NPAD // _NS
_CH = 64
_NRCH = _RPT // _CH
_U = 8
_NBODY = 40
_ECH = _U * _NBODY
_EPT = _ECH * _CH
_EPAD = _NS * _EPT
_EROWS = _EPAD // _CH


def _deg_body(src_ref, dst_ref, dego_ref, degi_ref, idxb, ones_v, stage_v,
              deg_sh, sdeg):
    cid = lax.axis_index("c")
    sid = lax.axis_index("s")

    def fill1(i, _):
        ones_v[pl.ds(i * _L, _L)] = jnp.ones((_L,), jnp.float32)
        return 0
    lax.fori_loop(0, _CH // _L, fill1, 0)

    def fill0(i, _):
        stage_v[pl.ds(i * _L, _L)] = jnp.zeros((_L,), jnp.float32)
        return 0
    lax.fori_loop(0, _RPT // _L, fill0, 0)
    pltpu.sync_copy(stage_v, deg_sh.at[pl.ds(sid * _RPT, _RPT)])
    plsc.subcore_barrier()

    def body(j, _):
        brow = sid * _ECH + j * _U

        @pl.when(cid == 0)
        def _():
            pltpu.sync_copy(src_ref.at[pl.ds(brow, _U)], idxb)

        @pl.when(cid == 1)
        def _():
            pltpu.sync_copy(dst_ref.at[pl.ds(brow, _U)], idxb)

        dss = [pltpu.async_copy(ones_v, deg_sh.at[idxb.at[k]], sdeg,
                                add=True)
               for k in range(_U)]
        for d in dss:
            d.wait()
        return 0
    lax.fori_loop(0, _NBODY, body, 0)
    plsc.subcore_barrier()

    pltpu.sync_copy(deg_sh.at[pl.ds(sid * _RPT, _RPT)], stage_v)

    @pl.when(cid == 0)
    def _():
        pltpu.sync_copy(stage_v, dego_ref.at[pl.ds(sid * _RPT, _RPT)])

    @pl.when(cid == 1)
    def _():
        pltpu.sync_copy(stage_v, degi_ref.at[pl.ds(sid * _RPT, _RPT)])


_PBLK = _NPAD // 8


def _prep_body(degT_ref, x_ref, cn_ref, nd_ref, g0_ref):
    dg = degT_ref[...]
    ns = lax.rsqrt(jnp.maximum(dg[:, 0:1], 1.0))
    ndv = lax.rsqrt(jnp.maximum(dg[:, 1:2], 1.0))
    cn_ref[...] = jnp.broadcast_to((1.0 - _ALPHA) * ns * ndv, (_PBLK, _D))
    nd_ref[...] = jnp.broadcast_to(ndv, (_PBLK, _D))
    g0_ref[...] = ns * x_ref[...]


def _loop_body(src_ref, dst_ref, g0_ref, x_ref, cn_ref, nd_ref,
               gw_ref, hg_ref, hk_ref,
               is0, id0, is1, id1, buf_a, buf_b, buf_c, buf_d, buf_e, agg_sh,
               si0, sd0, si1, sd1, sg0, sg1, sg2, sg3, ss0, ss1, ss2, ss3):
    cid = lax.axis_index("c")
    sid = lax.axis_index("s")
    row0 = sid * _RPT
    coff = cid * _NPAD
    sbase = cid * _EROWS + sid * _ECH

    def zrow(r, _):
        for c in range(_D // _L):
            buf_d[r, pl.ds(c * _L, _L)] = jnp.zeros((_L,), jnp.float32)
        return 0
    lax.fori_loop(0, _CH, zrow, 0)

    def init_chunk(rb, _):
        r = row0 + rb * _CH
        pltpu.sync_copy(g0_ref.at[pl.ds(coff + r, _CH)], buf_a)
        pltpu.sync_copy(buf_a, gw_ref.at[pl.ds(coff + r, _CH)])
        pltpu.sync_copy(buf_d, agg_sh.at[pl.ds(r, _CH)])
        return 0
    lax.fori_loop(0, _NRCH, init_chunk, 0)
    plsc.subcore_barrier()

    ring = (buf_a, buf_b, buf_c, buf_e)
    sgs = (sg0, sg1, sg2, sg3)
    sss = (ss0, ss1, ss2, ss3)
    ibufs = ((is0, id0, si0, sd0), (is1, id1, si1, sd1))

    def idx_start(jn, ib):
        isX, idX, siX, sdX = ib
        brow = jnp.minimum(sbase + jn * _U, 2 * _EROWS - _U)
        drow = jnp.minimum(sid * _ECH + jn * _U, _EROWS - _U)
        pltpu.async_copy(src_ref.at[pl.ds(brow, _U)], isX, siX)
        pltpu.async_copy(dst_ref.at[pl.ds(drow, _U)], idX, sdX)

    def idx_wait(ib):
        isX, idX, siX, sdX = ib
        pltpu.make_async_copy(src_ref.at[pl.ds(0, _U)], isX, siX).wait()
        pltpu.make_async_copy(dst_ref.at[pl.ds(0, _U)], idX, sdX).wait()

    def edge_phase():
        idx_start(0, ibufs[0])

        def super_body(sb, _):
            for p in range(2):
                j = 2 * sb + p
                ibX = ibufs[p]
                idx_wait(ibX)
                idx_start(j + 1, ibufs[1 - p])
                isX, idX = ibX[0], ibX[1]
                dgs = [None] * _U
                dss = [None] * _U
                for k in range(_U):
                    slot = k % 4
                    if k >= 4:
                        dss[k - 4].wait()
                    dgs[k] = pltpu.async_copy(gw_ref.at[isX.at[k]],
                                              ring[slot], sgs[slot])
                    if k >= 2:
                        dgs[k - 2].wait()
                        dss[k - 2] = pltpu.async_copy(
                            ring[(k - 2) % 4], agg_sh.at[idX.at[k - 2]],
                            sss[(k - 2) % 4], add=True)
                for k in (_U - 2, _U - 1):
                    dgs[k].wait()
                    dss[k] = pltpu.async_copy(
                        ring[k % 4], agg_sh.at[idX.at[k]], sss[k % 4],
                        add=True)
                for k in range(_U - 4, _U):
                    dss[k].wait()
            return 0
        lax.fori_loop(0, _NBODY // 2, super_body, 0)
        idx_wait(ibufs[0])

    def ew(fn):
        def row(r, _):
            for c in range(_D // _L):
                fn(r, c * _L)
            return 0
        lax.fori_loop(0, _CH, row, 0)

    for t in range(_K):
        edge_phase()
        plsc.subcore_barrier()

        last = (t == _K - 1)

        def dense_chunk(rb, _, t=t, last=last):
            r = row0 + rb * _CH
            rg = coff + r
            pltpu.sync_copy(agg_sh.at[pl.ds(r, _CH)], buf_a)
            if t == 0:
                pltpu.sync_copy(nd_ref.at[pl.ds(r, _CH)], buf_b)

                def body_hg(rr, cs):
                    buf_b[rr, pl.ds(cs, _L)] = (
                        buf_b[rr, pl.ds(cs, _L)] * buf_a[rr, pl.ds(cs, _L)])
                ew(body_hg)
                pltpu.sync_copy(buf_b, hg_ref.at[pl.ds(rg, _CH)])
            if not last:
                pltpu.sync_copy(cn_ref.at[pl.ds(r, _CH)], buf_b)
                pltpu.sync_copy(g0_ref.at[pl.ds(rg, _CH)], buf_c)

                def body_g(rr, cs):
                    buf_a[rr, pl.ds(cs, _L)] = (
                        buf_b[rr, pl.ds(cs, _L)] * buf_a[rr, pl.ds(cs, _L)]
                        + _ALPHA * buf_c[rr, pl.ds(cs, _L)])
                ew(body_g)
                pltpu.sync_copy(buf_a, gw_ref.at[pl.ds(rg, _CH)])
                pltpu.sync_copy(buf_d, agg_sh.at[pl.ds(r, _CH)])
            else:
                pltpu.sync_copy(nd_ref.at[pl.ds(r, _CH)], buf_b)
                pltpu.sync_copy(x_ref.at[pl.ds(rg, _CH)], buf_c)

                def body_hk(rr, cs):
                    buf_a[rr, pl.ds(cs, _L)] = (
                        (1.0 - _ALPHA)
                        * buf_b[rr, pl.ds(cs, _L)] * buf_a[rr, pl.ds(cs, _L)]
                        + _ALPHA * buf_c[rr, pl.ds(cs, _L)])
                ew(body_hk)
                pltpu.sync_copy(buf_a, hk_ref.at[pl.ds(rg, _CH)])
            return 0
        lax.fori_loop(0, _NRCH, dense_chunk, 0)
        plsc.subcore_barrier()


_NBLK = 2000


def _prelu(x, w):
    return jnp.where(x > 0, x, w * x)


def _sums_body(hgf_ref, hkf_ref, w1_ref, b1_ref, wl_ref, bl_ref, pw_ref,
               sums_ref):
    i = pl.program_id(0)
    w = pw_ref[0, 0]
    h1 = _prelu(jnp.dot(hgf_ref[...], w1_ref[...],
                        preferred_element_type=jnp.float32) + b1_ref[...], w)
    h2 = _prelu(jnp.dot(hkf_ref[...], wl_ref[...],
                        preferred_element_type=jnp.float32) + bl_ref[...], w)

    @pl.when(i == 0)
    def _():
        sums_ref[...] = jnp.zeros((8, _D), jnp.float32)

    sums_ref[0:1, :] = sums_ref[0:1, :] + jnp.sum(h1, axis=0, keepdims=True)
    sums_ref[1:2, :] = sums_ref[1:2, :] + jnp.sum(h2, axis=0, keepdims=True)


def _scores_body(sums_ref, hgf_ref, hkf_ref, hgs_ref, hks_ref,
                 w1_ref, b1_ref, wl_ref, bl_ref, wb_ref, bb_ref, pw_ref,
                 out_ref):
    w = pw_ref[0, 0]
    bb = bb_ref[0, 0]
    c1 = jax.nn.sigmoid(sums_ref[0:1, :] * (1.0 / _N))
    c2 = jax.nn.sigmoid(sums_ref[1:2, :] * (1.0 / _N))
    q1 = lax.dot_general(wb_ref[...], c1, (((1,), (1,)), ((), ())),
                         preferred_element_type=jnp.float32)
    q2 = lax.dot_general(wb_ref[...], c2, (((1,), (1,)), ((), ())),
                         preferred_element_type=jnp.float32)
    h1 = _prelu(jnp.dot(hgf_ref[...], w1_ref[...],
                        preferred_element_type=jnp.float32) + b1_ref[...], w)
    h2 = _prelu(jnp.dot(hkf_ref[...], wl_ref[...],
                        preferred_element_type=jnp.float32) + bl_ref[...], w)
    h3 = _prelu(jnp.dot(hgs_ref[...], w1_ref[...],
                        preferred_element_type=jnp.float32) + b1_ref[...], w)
    h4 = _prelu(jnp.dot(hks_ref[...], wl_ref[...],
                        preferred_element_type=jnp.float32) + bl_ref[...], w)
    out_ref[:, 0:1] = jnp.dot(h2, q1, preferred_element_type=jnp.float32) + bb
    out_ref[:, 1:2] = jnp.dot(h1, q2, preferred_element_type=jnp.float32) + bb
    out_ref[:, 2:3] = jnp.dot(h4, q1, preferred_element_type=jnp.float32) + bb
    out_ref[:, 3:4] = jnp.dot(h3, q2, preferred_element_type=jnp.float32) + bb


@functools.cache
def _build_calls():
    sc_mesh = plsc.VectorSubcoreMesh(core_axis_name="c", subcore_axis_name="s")
    deg_call = pl.kernel(
        _deg_body,
        out_type=(
            jax.ShapeDtypeStruct((_NPAD,), jnp.float32),
            jax.ShapeDtypeStruct((_NPAD,), jnp.float32),
        ),
        mesh=sc_mesh,
        scratch_types=[
            pltpu.VMEM((_U, _CH), jnp.int32),
            pltpu.VMEM((_CH,), jnp.float32),
            pltpu.VMEM((_RPT,), jnp.float32),
            pltpu.VMEM_SHARED((_NPAD,), jnp.float32),
            pltpu.SemaphoreType.DMA,
        ],
    )
    prep_call = pl.pallas_call(
        _prep_body,
        grid=(2, 8),
        in_specs=[
            pl.BlockSpec((_PBLK, 2), lambda c, b: (b, 0)),
            pl.BlockSpec((_PBLK, _D), lambda c, b: (c * 8 + b, 0)),
        ],
        out_specs=[
            pl.BlockSpec((_PBLK, _D), lambda c, b: (b, 0)),
            pl.BlockSpec((_PBLK, _D), lambda c, b: (b, 0)),
            pl.BlockSpec((_PBLK, _D), lambda c, b: (c * 8 + b, 0)),
        ],
        out_shape=[
            jax.ShapeDtypeStruct((_NPAD, _D), jnp.float32),
            jax.ShapeDtypeStruct((_NPAD, _D), jnp.float32),
            jax.ShapeDtypeStruct((2 * _NPAD, _D), jnp.float32),
        ],
    )
    loop_call = pl.kernel(
        _loop_body,
        out_type=(
            jax.ShapeDtypeStruct((2 * _NPAD, _D), jnp.float32),
            jax.ShapeDtypeStruct((2 * _NPAD, _D), jnp.float32),
            jax.ShapeDtypeStruct((2 * _NPAD, _D), jnp.float32),
        ),
        mesh=sc_mesh,
        scratch_types=(
            [pltpu.VMEM((_U, _CH), jnp.int32) for _ in range(4)]
            + [pltpu.VMEM((_CH, _D), jnp.float32) for _ in range(5)]
            + [pltpu.VMEM_SHARED((_NPAD, _D), jnp.float32)]
            + [pltpu.SemaphoreType.DMA for _ in range(12)]
        ),
    )
    sums_call = pl.pallas_call(
        _sums_body,
        grid=(_N // _NBLK,),
        in_specs=[
            pl.BlockSpec((_NBLK, _D), lambda b: (b, 0)),
            pl.BlockSpec((_NBLK, _D), lambda b: (b, 0)),
            pl.BlockSpec((_D, _D), lambda b: (0, 0)),
            pl.BlockSpec((1, _D), lambda b: (0, 0)),
            pl.BlockSpec((_D, _D), lambda b: (0, 0)),
            pl.BlockSpec((1, _D), lambda b: (0, 0)),
            pl.BlockSpec((1, 1), lambda b: (0, 0)),
        ],
        out_specs=pl.BlockSpec((8, _D), lambda b: (0, 0)),
        out_shape=jax.ShapeDtypeStruct((8, _D), jnp.float32),
    )
    scores_call = pl.pallas_call(
        _scores_body,
        grid=(_N // _NBLK,),
        in_specs=[
            pl.BlockSpec((8, _D), lambda b: (0, 0)),
            pl.BlockSpec((_NBLK, _D), lambda b: (b, 0)),
            pl.BlockSpec((_NBLK, _D), lambda b: (b, 0)),
            pl.BlockSpec((_NBLK, _D), lambda b: (b, 0)),
            pl.BlockSpec((_NBLK, _D), lambda b: (b, 0)),
            pl.BlockSpec((_D, _D), lambda b: (0, 0)),
            pl.BlockSpec((1, _D), lambda b: (0, 0)),
            pl.BlockSpec((_D, _D), lambda b: (0, 0)),
            pl.BlockSpec((1, _D), lambda b: (0, 0)),
            pl.BlockSpec((_D, _D), lambda b: (0, 0)),
            pl.BlockSpec((1, 1), lambda b: (0, 0)),
            pl.BlockSpec((1, 1), lambda b: (0, 0)),
        ],
        out_specs=pl.BlockSpec((_NBLK, 4), lambda b: (b, 0)),
        out_shape=jax.ShapeDtypeStruct((_N, 4), jnp.float32),
    )
    return deg_call, prep_call, loop_call, sums_call, scores_call


def kernel(feat, shuf_feat, edge_index, W1, b1, Wlin, blin, Wbil, bbil,
           prelu_w):
    deg_call, prep_call, loop_call, sums_call, scores_call = _build_calls()

    src = edge_index[0].astype(jnp.int32)
    dst = edge_index[1].astype(jnp.int32)
    pad_ids = _N + (jnp.arange(_EPAD - _E, dtype=jnp.int32) % (_NPAD - _N))
    src1 = jnp.concatenate([src, pad_ids]).reshape(_EROWS, _CH)
    srcp = jnp.concatenate([src1, src1 + _NPAD], axis=0)
    dstp = jnp.concatenate([dst, pad_ids]).reshape(_EROWS, _CH)

    xf = jnp.pad(feat, ((0, _NPAD - _N), (0, 0)))
    xs = jnp.pad(shuf_feat, ((0, _NPAD - _N), (0, 0)))
    xcat = jnp.concatenate([xf, xs], axis=0)

    dego, degi = deg_call(srcp, dstp)
    degT = jnp.stack([dego, degi], axis=1)
    cn, nd, g0cat = prep_call(degT, xcat)
    _, hgcat, hkcat = loop_call(srcp, dstp, g0cat, xcat, cn, nd)

    hgf = hgcat[:_N]
    hgs = hgcat[_NPAD:_NPAD + _N]
    hkf = hkcat[:_N]
    hks = hkcat[_NPAD:_NPAD + _N]

    b1r = b1.reshape(1, _D)
    blr = blin.reshape(1, _D)
    pwr = prelu_w.reshape(1, 1)
    bbr = bbil.reshape(1, 1)

    sums = sums_call(hgf, hkf, W1, b1r, Wlin, blr, pwr)
    scores = scores_call(sums, hgf, hkf, hgs, hks, W1, b1r, Wlin, blr,
                         Wbil, bbr, pwr)
    return scores.T.reshape(4 * _N)


# speedup vs baseline: 9.4792x; 1.0089x over previous
import functools

import jax
import jax.numpy as jnp
from jax import lax
from jax.experimental import pallas as pl
from jax.experimental.pallas import tpu as pltpu
from jax.experimental.pallas import tpu_sc as plsc

_N = 10000
_E = 320000
_D = 128
_K = 10
_ALPHA = 0.1

_NS = 16
_L = 16

_NPAD = 10240
_RPT = _NPAD // _NS
_CH = 64
_NRCH = _RPT // _CH
_SC = 32
_UB = 16
_NBODY = 40
_EPT = _UB * _SC * _NBODY
_EPAD = _NS * _EPT
_EROWS = _EPAD // _SC
_TROWS = _EPT // _SC


def _deg_body(src_ref, dst_ref, dego_ref, degi_ref, idxb, ones_v, stage_v,
              deg_sh, sdeg):
    cid = lax.axis_index("c")
    sid = lax.axis_index("s")

    def fill1(i, _):
        ones_v[pl.ds(i * _L, _L)] = jnp.ones((_L,), jnp.float32)
        return 0
    lax.fori_loop(0, _SC // _L, fill1, 0)

    def fill0(i, _):
        stage_v[pl.ds(i * _L, _L)] = jnp.zeros((_L,), jnp.float32)
        return 0
    lax.fori_loop(0, _RPT // _L, fill0, 0)
    pltpu.sync_copy(stage_v, deg_sh.at[pl.ds(sid * _RPT, _RPT)])
    plsc.subcore_barrier()

    def body(j, _):
        brow = sid * _TROWS + j * _UB

        @pl.when(cid == 0)
        def _():
            pltpu.sync_copy(src_ref.at[pl.ds(brow, _UB)], idxb)

        @pl.when(cid == 1)
        def _():
            pltpu.sync_copy(dst_ref.at[pl.ds(brow, _UB)], idxb)

        dss = [pltpu.async_copy(ones_v, deg_sh.at[idxb.at[k]], sdeg,
                                add=True)
               for k in range(_UB)]
        for d in dss:
            d.wait()
        return 0
    lax.fori_loop(0, _NBODY, body, 0)
    plsc.subcore_barrier()

    pltpu.sync_copy(deg_sh.at[pl.ds(sid * _RPT, _RPT)], stage_v)

    @pl.when(cid == 0)
    def _():
        pltpu.sync_copy(stage_v, dego_ref.at[pl.ds(sid * _RPT, _RPT)])

    @pl.when(cid == 1)
    def _():
        pltpu.sync_copy(stage_v, degi_ref.at[pl.ds(sid * _RPT, _RPT)])


_PBLK = _NPAD // 8


def _prep_body(degT_ref, x_ref, cn_ref, nd_ref, g0_ref):
    dg = degT_ref[...]
    ns = lax.rsqrt(jnp.maximum(dg[:, 0:1], 1.0))
    ndv = lax.rsqrt(jnp.maximum(dg[:, 1:2], 1.0))
    cn_ref[...] = jnp.broadcast_to((1.0 - _ALPHA) * ns * ndv, (_PBLK, _D))
    nd_ref[...] = jnp.broadcast_to(ndv, (_PBLK, _D))
    g0_ref[...] = ns * x_ref[...]


def _loop_body(src_ref, dst_ref, g0_ref, x_ref, cn_ref, nd_ref,
               gw_ref, hg_ref, hk_ref,
               is0, id0, is1, id1, buf_a, buf_b, buf_c, buf_d, buf_e, agg_sh,
               *sems):
    cid = lax.axis_index("c")
    sid = lax.axis_index("s")
    row0 = sid * _RPT
    coff = cid * _NPAD
    sbase = cid * _EROWS + sid * _TROWS
    si0, sd0, si1, sd1 = sems[0:4]
    sgs = sems[4:12]
    sss = sems[12:20]

    def zrow(r, _):
        for c in range(_D // _L):
            buf_d[r, pl.ds(c * _L, _L)] = jnp.zeros((_L,), jnp.float32)
        return 0
    lax.fori_loop(0, _CH, zrow, 0)

    def init_chunk(rb, _):
        r = row0 + rb * _CH
        pltpu.sync_copy(g0_ref.at[pl.ds(coff + r, _CH)], buf_a)
        pltpu.sync_copy(buf_a, gw_ref.at[pl.ds(coff + r, _CH)])
        pltpu.sync_copy(buf_d, agg_sh.at[pl.ds(r, _CH)])
        return 0
    lax.fori_loop(0, _NRCH, init_chunk, 0)
    plsc.subcore_barrier()

    _ring = (buf_a, buf_b, buf_c, buf_e)

    def slot_ref(s):
        return _ring[s // 2].at[pl.ds((s % 2) * _SC, _SC)]

    ibufs = ((is0, id0, si0, sd0), (is1, id1, si1, sd1))

    def idx_start(jn, ib):
        isX, idX, siX, sdX = ib
        brow = jnp.minimum(sbase + jn * _UB, 2 * _EROWS - _UB)
        drow = jnp.minimum(sid * _TROWS + jn * _UB, _EROWS - _UB)
        pltpu.async_copy(src_ref.at[pl.ds(brow, _UB)], isX, siX)
        pltpu.async_copy(dst_ref.at[pl.ds(drow, _UB)], idX, sdX)

    def idx_wait(ib):
        isX, idX, siX, sdX = ib
        pltpu.make_async_copy(src_ref.at[pl.ds(0, _UB)], isX, siX).wait()
        pltpu.make_async_copy(dst_ref.at[pl.ds(0, _UB)], idX, sdX).wait()

    def edge_phase():
        idx_start(0, ibufs[0])

        def super_body(sb, _):
            for p in range(2):
                j = 2 * sb + p
                ibX = ibufs[p]
                idx_wait(ibX)
                idx_start(j + 1, ibufs[1 - p])
                isX, idX = ibX[0], ibX[1]
                dgs = [None] * _UB
                dss = [None] * _UB
                for k in range(_UB):
                    slot = k % 8
                    if k >= 8:
                        dss[k - 8].wait()
                    dgs[k] = pltpu.async_copy(gw_ref.at[isX.at[k]],
                                              slot_ref(slot), sgs[slot])
                    if k >= 4:
                        dgs[k - 4].wait()
                        dss[k - 4] = pltpu.async_copy(
                            slot_ref((k - 4) % 8), agg_sh.at[idX.at[k - 4]],
                            sss[(k - 4) % 8], add=True)
                for k in range(_UB - 4, _UB):
                    dgs[k].wait()
                    dss[k] = pltpu.async_copy(
                        slot_ref(k % 8), agg_sh.at[idX.at[k]], sss[k % 8],
                        add=True)
                for k in range(_UB - 8, _UB):
                    dss[k].wait()
            return 0
        lax.fori_loop(0, _NBODY // 2, super_body, 0)
        idx_wait(ibufs[0])

    def ew(fn):
        def row(r, _):
            for c in range(_D // _L):
                fn(r, c * _L)
            return 0
        lax.fori_loop(0, _CH, row, 0)

    for t in range(_K):
        edge_phase()
        plsc.subcore_barrier()

        last = (t == _K - 1)

        def dense_chunk(rb, _, t=t, last=last):
            r = row0 + rb * _CH
            rg = coff + r
            pltpu.sync_copy(agg_sh.at[pl.ds(r, _CH)], buf_a)
            if t == 0:
                pltpu.sync_copy(nd_ref.at[pl.ds(r, _CH)], buf_b)

                def body_hg(rr, cs):
                    buf_b[rr, pl.ds(cs, _L)] = (
                        buf_b[rr, pl.ds(cs, _L)] * buf_a[rr, pl.ds(cs, _L)])
                ew(body_hg)
                pltpu.sync_copy(buf_b, hg_ref.at[pl.ds(rg, _CH)])
            if not last:
                pltpu.sync_copy(cn_ref.at[pl.ds(r, _CH)], buf_b)
                pltpu.sync_copy(g0_ref.at[pl.ds(rg, _CH)], buf_c)

                def body_g(rr, cs):
                    buf_a[rr, pl.ds(cs, _L)] = (
                        buf_b[rr, pl.ds(cs, _L)] * buf_a[rr, pl.ds(cs, _L)]
                        + _ALPHA * buf_c[rr, pl.ds(cs, _L)])
                ew(body_g)
                pltpu.sync_copy(buf_a, gw_ref.at[pl.ds(rg, _CH)])
                pltpu.sync_copy(buf_d, agg_sh.at[pl.ds(r, _CH)])
            else:
                pltpu.sync_copy(nd_ref.at[pl.ds(r, _CH)], buf_b)
                pltpu.sync_copy(x_ref.at[pl.ds(rg, _CH)], buf_c)

                def body_hk(rr, cs):
                    buf_a[rr, pl.ds(cs, _L)] = (
                        (1.0 - _ALPHA)
                        * buf_b[rr, pl.ds(cs, _L)] * buf_a[rr, pl.ds(cs, _L)]
                        + _ALPHA * buf_c[rr, pl.ds(cs, _L)])
                ew(body_hk)
                pltpu.sync_copy(buf_a, hk_ref.at[pl.ds(rg, _CH)])
            return 0
        lax.fori_loop(0, _NRCH, dense_chunk, 0)
        plsc.subcore_barrier()


_NBLK = 2000


def _prelu(x, w):
    return jnp.where(x > 0, x, w * x)


def _sums_body(hgf_ref, hkf_ref, w1_ref, b1_ref, wl_ref, bl_ref, pw_ref,
               sums_ref):
    i = pl.program_id(0)
    w = pw_ref[0, 0]
    h1 = _prelu(jnp.dot(hgf_ref[...], w1_ref[...],
                        preferred_element_type=jnp.float32) + b1_ref[...], w)
    h2 = _prelu(jnp.dot(hkf_ref[...], wl_ref[...],
                        preferred_element_type=jnp.float32) + bl_ref[...], w)

    @pl.when(i == 0)
    def _():
        sums_ref[...] = jnp.zeros((8, _D), jnp.float32)

    sums_ref[0:1, :] = sums_ref[0:1, :] + jnp.sum(h1, axis=0, keepdims=True)
    sums_ref[1:2, :] = sums_ref[1:2, :] + jnp.sum(h2, axis=0, keepdims=True)


def _scores_body(sums_ref, hgf_ref, hkf_ref, hgs_ref, hks_ref,
                 w1_ref, b1_ref, wl_ref, bl_ref, wb_ref, bb_ref, pw_ref,
                 out_ref):
    w = pw_ref[0, 0]
    bb = bb_ref[0, 0]
    c1 = jax.nn.sigmoid(sums_ref[0:1, :] * (1.0 / _N))
    c2 = jax.nn.sigmoid(sums_ref[1:2, :] * (1.0 / _N))
    q1 = lax.dot_general(wb_ref[...], c1, (((1,), (1,)), ((), ())),
                         preferred_element_type=jnp.float32)
    q2 = lax.dot_general(wb_ref[...], c2, (((1,), (1,)), ((), ())),
                         preferred_element_type=jnp.float32)
    h1 = _prelu(jnp.dot(hgf_ref[...], w1_ref[...],
                        preferred_element_type=jnp.float32) + b1_ref[...], w)
    h2 = _prelu(jnp.dot(hkf_ref[...], wl_ref[...],
                        preferred_element_type=jnp.float32) + bl_ref[...], w)
    h3 = _prelu(jnp.dot(hgs_ref[...], w1_ref[...],
                        preferred_element_type=jnp.float32) + b1_ref[...], w)
    h4 = _prelu(jnp.dot(hks_ref[...], wl_ref[...],
                        preferred_element_type=jnp.float32) + bl_ref[...], w)
    out_ref[:, 0:1] = jnp.dot(h2, q1, preferred_element_type=jnp.float32) + bb
    out_ref[:, 1:2] = jnp.dot(h1, q2, preferred_element_type=jnp.float32) + bb
    out_ref[:, 2:3] = jnp.dot(h4, q1, preferred_element_type=jnp.float32) + bb
    out_ref[:, 3:4] = jnp.dot(h3, q2, preferred_element_type=jnp.float32) + bb


@functools.cache
def _build_calls():
    sc_mesh = plsc.VectorSubcoreMesh(core_axis_name="c", subcore_axis_name="s")
    deg_call = pl.kernel(
        _deg_body,
        out_type=(
            jax.ShapeDtypeStruct((_NPAD,), jnp.float32),
            jax.ShapeDtypeStruct((_NPAD,), jnp.float32),
        ),
        mesh=sc_mesh,
        scratch_types=[
            pltpu.VMEM((_UB, _SC), jnp.int32),
            pltpu.VMEM((_SC,), jnp.float32),
            pltpu.VMEM((_RPT,), jnp.float32),
            pltpu.VMEM_SHARED((_NPAD,), jnp.float32),
            pltpu.SemaphoreType.DMA,
        ],
    )
    prep_call = pl.pallas_call(
        _prep_body,
        grid=(2, 8),
        in_specs=[
            pl.BlockSpec((_PBLK, 2), lambda c, b: (b, 0)),
            pl.BlockSpec((_PBLK, _D), lambda c, b: (c * 8 + b, 0)),
        ],
        out_specs=[
            pl.BlockSpec((_PBLK, _D), lambda c, b: (b, 0)),
            pl.BlockSpec((_PBLK, _D), lambda c, b: (b, 0)),
            pl.BlockSpec((_PBLK, _D), lambda c, b: (c * 8 + b, 0)),
        ],
        out_shape=[
            jax.ShapeDtypeStruct((_NPAD, _D), jnp.float32),
            jax.ShapeDtypeStruct((_NPAD, _D), jnp.float32),
            jax.ShapeDtypeStruct((2 * _NPAD, _D), jnp.float32),
        ],
    )
    loop_call = pl.kernel(
        _loop_body,
        out_type=(
            jax.ShapeDtypeStruct((2 * _NPAD, _D), jnp.float32),
            jax.ShapeDtypeStruct((2 * _NPAD, _D), jnp.float32),
            jax.ShapeDtypeStruct((2 * _NPAD, _D), jnp.float32),
        ),
        mesh=sc_mesh,
        scratch_types=(
            [pltpu.VMEM((_UB, _SC), jnp.int32) for _ in range(4)]
            + [pltpu.VMEM((_CH, _D), jnp.float32) for _ in range(5)]
            + [pltpu.VMEM_SHARED((_NPAD, _D), jnp.float32)]
            + [pltpu.SemaphoreType.DMA for _ in range(20)]
        ),
    )
    sums_call = pl.pallas_call(
        _sums_body,
        grid=(_N // _NBLK,),
        in_specs=[
            pl.BlockSpec((_NBLK, _D), lambda b: (b, 0)),
            pl.BlockSpec((_NBLK, _D), lambda b: (b, 0)),
            pl.BlockSpec((_D, _D), lambda b: (0, 0)),
            pl.BlockSpec((1, _D), lambda b: (0, 0)),
            pl.BlockSpec((_D, _D), lambda b: (0, 0)),
            pl.BlockSpec((1, _D), lambda b: (0, 0)),
            pl.BlockSpec((1, 1), lambda b: (0, 0)),
        ],
        out_specs=pl.BlockSpec((8, _D), lambda b: (0, 0)),
        out_shape=jax.ShapeDtypeStruct((8, _D), jnp.float32),
    )
    scores_call = pl.pallas_call(
        _scores_body,
        grid=(_N // _NBLK,),
        in_specs=[
            pl.BlockSpec((8, _D), lambda b: (0, 0)),
            pl.BlockSpec((_NBLK, _D), lambda b: (b, 0)),
            pl.BlockSpec((_NBLK, _D), lambda b: (b, 0)),
            pl.BlockSpec((_NBLK, _D), lambda b: (b, 0)),
            pl.BlockSpec((_NBLK, _D), lambda b: (b, 0)),
            pl.BlockSpec((_D, _D), lambda b: (0, 0)),
            pl.BlockSpec((1, _D), lambda b: (0, 0)),
            pl.BlockSpec((_D, _D), lambda b: (0, 0)),
            pl.BlockSpec((1, _D), lambda b: (0, 0)),
            pl.BlockSpec((_D, _D), lambda b: (0, 0)),
            pl.BlockSpec((1, 1), lambda b: (0, 0)),
            pl.BlockSpec((1, 1), lambda b: (0, 0)),
        ],
        out_specs=pl.BlockSpec((_NBLK, 4), lambda b: (b, 0)),
        out_shape=jax.ShapeDtypeStruct((_N, 4), jnp.float32),
    )
    return deg_call, prep_call, loop_call, sums_call, scores_call


def kernel(feat, shuf_feat, edge_index, W1, b1, Wlin, blin, Wbil, bbil,
           prelu_w):
    deg_call, prep_call, loop_call, sums_call, scores_call = _build_calls()

    src = edge_index[0].astype(jnp.int32)
    dst = edge_index[1].astype(jnp.int32)
    pad_ids = _N + (jnp.arange(_EPAD - _E, dtype=jnp.int32) % (_NPAD - _N))
    src1 = jnp.concatenate([src, pad_ids]).reshape(_EROWS, _SC)
    srcp = jnp.concatenate([src1, src1 + _NPAD], axis=0)
    dstp = jnp.concatenate([dst, pad_ids]).reshape(_EROWS, _SC)

    xf = jnp.pad(feat, ((0, _NPAD - _N), (0, 0)))
    xs = jnp.pad(shuf_feat, ((0, _NPAD - _N), (0, 0)))
    xcat = jnp.concatenate([xf, xs], axis=0)

    dego, degi = deg_call(srcp, dstp)
    degT = jnp.stack([dego, degi], axis=1)
    cn, nd, g0cat = prep_call(degT, xcat)
    _, hgcat, hkcat = loop_call(srcp, dstp, g0cat, xcat, cn, nd)

    hgf = hgcat[:_N]
    hgs = hgcat[_NPAD:_NPAD + _N]
    hkf = hkcat[:_N]
    hks = hkcat[_NPAD:_NPAD + _N]

    b1r = b1.reshape(1, _D)
    blr = blin.reshape(1, _D)
    pwr = prelu_w.reshape(1, 1)
    bbr = bbil.reshape(1, 1)

    sums = sums_call(hgf, hkf, W1, b1r, Wlin, blr, pwr)
    scores = scores_call(sums, hgf, hkf, hgs, hks, W1, b1r, Wlin, blr,
                         Wbil, bbr, pwr)
    return scores.T.reshape(4 * _N)


# speedup vs baseline: 10.2273x; 1.0789x over previous
import functools

import jax
import jax.numpy as jnp
from jax import lax
from jax.experimental import pallas as pl
from jax.experimental.pallas import tpu as pltpu
from jax.experimental.pallas import tpu_sc as plsc

_N = 10000
_E = 320000
_D = 128
_K = 10
_ALPHA = 0.1

_NS = 16
_L = 16

_NPAD = 10240
_RPT = _NPAD // _NS
_RCH = 128
_NR = _RPT // _RCH
_CH = 128
_CPB = 16
_NBODY = 10
_EPT = _CPB * _CH * _NBODY
_EPAD = _NS * _EPT
_EROWS = _EPAD // _CH
_TROWS = _EPT // _CH


def _deg_body(src_ref, dst_ref, dego_ref, degi_ref, idxb, ones_v, stage_v,
              deg_sh, sdeg):
    cid = lax.axis_index("c")
    sid = lax.axis_index("s")

    def fill1(i, _):
        ones_v[pl.ds(i * _L, _L)] = jnp.ones((_L,), jnp.float32)
        return 0
    lax.fori_loop(0, _CH // _L, fill1, 0)

    def fill0(i, _):
        stage_v[pl.ds(i * _L, _L)] = jnp.zeros((_L,), jnp.float32)
        return 0
    lax.fori_loop(0, _RPT // _L, fill0, 0)
    pltpu.sync_copy(stage_v, deg_sh.at[pl.ds(sid * _RPT, _RPT)])
    plsc.subcore_barrier()

    def body(j, _):
        brow = sid * _TROWS + j * _CPB

        @pl.when(cid == 0)
        def _():
            pltpu.sync_copy(src_ref.at[pl.ds(brow, _CPB)], idxb)

        @pl.when(cid == 1)
        def _():
            pltpu.sync_copy(dst_ref.at[pl.ds(brow, _CPB)], idxb)

        dss = [pltpu.async_copy(ones_v, deg_sh.at[idxb.at[k]], sdeg,
                                add=True)
               for k in range(_CPB)]
        for d in dss:
            d.wait()
        return 0
    lax.fori_loop(0, _NBODY, body, 0)
    plsc.subcore_barrier()

    pltpu.sync_copy(deg_sh.at[pl.ds(sid * _RPT, _RPT)], stage_v)

    @pl.when(cid == 0)
    def _():
        pltpu.sync_copy(stage_v, dego_ref.at[pl.ds(sid * _RPT, _RPT)])

    @pl.when(cid == 1)
    def _():
        pltpu.sync_copy(stage_v, degi_ref.at[pl.ds(sid * _RPT, _RPT)])


_PBLK = _NPAD // 8


def _prep_body(degT_ref, x_ref, cn_ref, nd_ref, g0_ref):
    dg = degT_ref[...]
    ns = lax.rsqrt(jnp.maximum(dg[:, 0:1], 1.0))
    ndv = lax.rsqrt(jnp.maximum(dg[:, 1:2], 1.0))
    cn_ref[...] = jnp.broadcast_to((1.0 - _ALPHA) * ns * ndv, (_PBLK, _D))
    nd_ref[...] = jnp.broadcast_to(ndv, (_PBLK, _D))
    g0_ref[...] = ns * x_ref[...]


def _loop_body(src_ref, dst_ref, g0_ref, x_ref, cn_ref, nd_ref, zer_ref,
               gw_ref, hg_ref, hk_ref,
               is0, id0, is1, id1, bufp, bufq, agg_sh,
               si0, sd0, si1, sd1, sg0, sg1, ss0, ss1):
    cid = lax.axis_index("c")
    sid = lax.axis_index("s")
    row0 = sid * _RPT
    coff = cid * _NPAD
    sbase = cid * _EROWS + sid * _TROWS
    sgs = (sg0, sg1)
    sss = (ss0, ss1)
    bufs = (bufp, bufq)

    def init_chunk(rb, _):
        r = row0 + rb * _RCH
        pltpu.sync_copy(g0_ref.at[pl.ds(coff + r, _RCH)], bufp)
        pltpu.sync_copy(bufp, gw_ref.at[pl.ds(coff + r, _RCH)])
        pltpu.sync_copy(zer_ref, agg_sh.at[pl.ds(r, _RCH)])
        return 0
    lax.fori_loop(0, _NR, init_chunk, 0)
    plsc.subcore_barrier()

    ibufs = ((is0, id0, si0, sd0), (is1, id1, si1, sd1))

    def idx_start(jn, ib):
        isX, idX, siX, sdX = ib
        brow = jnp.minimum(sbase + jn * _CPB, 2 * _EROWS - _CPB)
        drow = jnp.minimum(sid * _TROWS + jn * _CPB, _EROWS - _CPB)
        pltpu.async_copy(src_ref.at[pl.ds(brow, _CPB)], isX, siX)
        pltpu.async_copy(dst_ref.at[pl.ds(drow, _CPB)], idX, sdX)

    def idx_wait(ib):
        isX, idX, siX, sdX = ib
        pltpu.make_async_copy(src_ref.at[pl.ds(0, _CPB)], isX, siX).wait()
        pltpu.make_async_copy(dst_ref.at[pl.ds(0, _CPB)], idX, sdX).wait()

    def edge_phase():
        idx_start(0, ibufs[0])

        def super_body(sb, _):
            for p in range(2):
                j = 2 * sb + p
                ibX = ibufs[p]
                idx_wait(ibX)
                idx_start(j + 1, ibufs[1 - p])
                isX, idX = ibX[0], ibX[1]
                dgs = [None] * _CPB
                dss = [None] * _CPB
                for k in range(_CPB):
                    if k >= 2:
                        dss[k - 2].wait()
                    dgs[k] = pltpu.async_copy(gw_ref.at[isX.at[k]],
                                              bufs[k % 2], sgs[k % 2])
                    if k >= 1:
                        dgs[k - 1].wait()
                        dss[k - 1] = pltpu.async_copy(
                            bufs[(k - 1) % 2], agg_sh.at[idX.at[k - 1]],
                            sss[(k - 1) % 2], add=True)
                km = _CPB - 1
                dgs[km].wait()
                dss[km] = pltpu.async_copy(
                    bufs[km % 2], agg_sh.at[idX.at[km]], sss[km % 2],
                    add=True)
                dss[km - 1].wait()
                dss[km].wait()
            return 0
        lax.fori_loop(0, _NBODY // 2, super_body, 0)
        idx_wait(ibufs[0])

    def ew(rows, fn):
        def rowfn(rr, _):
            for cc in range(_D // _L):
                fn(rr, pl.ds(cc * _L, _L))
            return 0
        lax.fori_loop(0, rows, rowfn, 0)

    def dense_mid(rb, _):
        r = row0 + rb * _RCH
        rg = coff + r
        pltpu.sync_copy(agg_sh.at[pl.ds(r, _RCH)], bufp)
        pltpu.sync_copy(cn_ref.at[pl.ds(r, _RCH)], bufq)

        def mul(rr, sl):
            bufp[rr, sl] = bufp[rr, sl] * bufq[rr, sl]
        ew(_RCH, mul)
        pltpu.sync_copy(g0_ref.at[pl.ds(rg, _RCH)], bufq)

        def axpy(rr, sl):
            bufp[rr, sl] = bufp[rr, sl] + _ALPHA * bufq[rr, sl]
        ew(_RCH, axpy)
        pltpu.sync_copy(bufp, gw_ref.at[pl.ds(rg, _RCH)])
        pltpu.sync_copy(zer_ref, agg_sh.at[pl.ds(r, _RCH)])
        return 0

    def dense_t0(rb, _):
        h = _RCH // 2
        r = row0 + rb * h
        rg = coff + r
        pltpu.sync_copy(agg_sh.at[pl.ds(r, h)], bufp.at[pl.ds(0, h)])
        pltpu.sync_copy(nd_ref.at[pl.ds(r, h)], bufq.at[pl.ds(0, h)])

        def hgmul(rr, sl):
            bufq[rr, sl] = bufq[rr, sl] * bufp[rr, sl]
        ew(h, hgmul)
        pltpu.sync_copy(bufq.at[pl.ds(0, h)], hg_ref.at[pl.ds(rg, h)])
        pltpu.sync_copy(cn_ref.at[pl.ds(r, h)], bufq.at[pl.ds(0, h)])

        def mul(rr, sl):
            bufp[rr, sl] = bufp[rr, sl] * bufq[rr, sl]
        ew(h, mul)
        pltpu.sync_copy(g0_ref.at[pl.ds(rg, h)], bufq.at[pl.ds(0, h)])

        def axpy(rr, sl):
            bufp[rr, sl] = bufp[rr, sl] + _ALPHA * bufq[rr, sl]
        ew(h, axpy)
        pltpu.sync_copy(bufp.at[pl.ds(0, h)], gw_ref.at[pl.ds(rg, h)])
        pltpu.sync_copy(zer_ref.at[pl.ds(0, h)], agg_sh.at[pl.ds(r, h)])
        return 0

    def dense_t9(rb, _):
        r = row0 + rb * _RCH
        rg = coff + r
        pltpu.sync_copy(agg_sh.at[pl.ds(r, _RCH)], bufp)
        pltpu.sync_copy(nd_ref.at[pl.ds(r, _RCH)], bufq)

        def mul9(rr, sl):
            bufp[rr, sl] = (1.0 - _ALPHA) * bufp[rr, sl] * bufq[rr, sl]
        ew(_RCH, mul9)
        pltpu.sync_copy(x_ref.at[pl.ds(rg, _RCH)], bufq)

        def axpy(rr, sl):
            bufp[rr, sl] = bufp[rr, sl] + _ALPHA * bufq[rr, sl]
        ew(_RCH, axpy)
        pltpu.sync_copy(bufp, hk_ref.at[pl.ds(rg, _RCH)])
        return 0

    for t in range(_K):
        edge_phase()
        plsc.subcore_barrier()
        if t == 0:
            lax.fori_loop(0, 2 * _NR, dense_t0, 0)
        elif t == _K - 1:
            lax.fori_loop(0, _NR, dense_t9, 0)
        else:
            lax.fori_loop(0, _NR, dense_mid, 0)
        plsc.subcore_barrier()


_NBLK = 2000


def _prelu(x, w):
    return jnp.where(x > 0, x, w * x)


def _sums_body(hgf_ref, hkf_ref, w1_ref, b1_ref, wl_ref, bl_ref, pw_ref,
               sums_ref):
    i = pl.program_id(0)
    w = pw_ref[0, 0]
    h1 = _prelu(jnp.dot(hgf_ref[...], w1_ref[...],
                        preferred_element_type=jnp.float32) + b1_ref[...], w)
    h2 = _prelu(jnp.dot(hkf_ref[...], wl_ref[...],
                        preferred_element_type=jnp.float32) + bl_ref[...], w)

    @pl.when(i == 0)
    def _():
        sums_ref[...] = jnp.zeros((8, _D), jnp.float32)

    sums_ref[0:1, :] = sums_ref[0:1, :] + jnp.sum(h1, axis=0, keepdims=True)
    sums_ref[1:2, :] = sums_ref[1:2, :] + jnp.sum(h2, axis=0, keepdims=True)


def _scores_body(sums_ref, hgf_ref, hkf_ref, hgs_ref, hks_ref,
                 w1_ref, b1_ref, wl_ref, bl_ref, wb_ref, bb_ref, pw_ref,
                 out_ref):
    w = pw_ref[0, 0]
    bb = bb_ref[0, 0]
    c1 = jax.nn.sigmoid(sums_ref[0:1, :] * (1.0 / _N))
    c2 = jax.nn.sigmoid(sums_ref[1:2, :] * (1.0 / _N))
    q1 = lax.dot_general(wb_ref[...], c1, (((1,), (1,)), ((), ())),
                         preferred_element_type=jnp.float32)
    q2 = lax.dot_general(wb_ref[...], c2, (((1,), (1,)), ((), ())),
                         preferred_element_type=jnp.float32)
    h1 = _prelu(jnp.dot(hgf_ref[...], w1_ref[...],
                        preferred_element_type=jnp.float32) + b1_ref[...], w)
    h2 = _prelu(jnp.dot(hkf_ref[...], wl_ref[...],
                        preferred_element_type=jnp.float32) + bl_ref[...], w)
    h3 = _prelu(jnp.dot(hgs_ref[...], w1_ref[...],
                        preferred_element_type=jnp.float32) + b1_ref[...], w)
    h4 = _prelu(jnp.dot(hks_ref[...], wl_ref[...],
                        preferred_element_type=jnp.float32) + bl_ref[...], w)
    out_ref[:, 0:1] = jnp.dot(h2, q1, preferred_element_type=jnp.float32) + bb
    out_ref[:, 1:2] = jnp.dot(h1, q2, preferred_element_type=jnp.float32) + bb
    out_ref[:, 2:3] = jnp.dot(h4, q1, preferred_element_type=jnp.float32) + bb
    out_ref[:, 3:4] = jnp.dot(h3, q2, preferred_element_type=jnp.float32) + bb


@functools.cache
def _build_calls():
    sc_mesh = plsc.VectorSubcoreMesh(core_axis_name="c", subcore_axis_name="s")
    deg_call = pl.kernel(
        _deg_body,
        out_type=(
            jax.ShapeDtypeStruct((_NPAD,), jnp.float32),
            jax.ShapeDtypeStruct((_NPAD,), jnp.float32),
        ),
        mesh=sc_mesh,
        scratch_types=[
            pltpu.VMEM((_CPB, _CH), jnp.int32),
            pltpu.VMEM((_CH,), jnp.float32),
            pltpu.VMEM((_RPT,), jnp.float32),
            pltpu.VMEM_SHARED((_NPAD,), jnp.float32),
            pltpu.SemaphoreType.DMA,
        ],
    )
    prep_call = pl.pallas_call(
        _prep_body,
        grid=(2, 8),
        in_specs=[
            pl.BlockSpec((_PBLK, 2), lambda c, b: (b, 0)),
            pl.BlockSpec((_PBLK, _D), lambda c, b: (c * 8 + b, 0)),
        ],
        out_specs=[
            pl.BlockSpec((_PBLK, _D), lambda c, b: (b, 0)),
            pl.BlockSpec((_PBLK, _D), lambda c, b: (b, 0)),
            pl.BlockSpec((_PBLK, _D), lambda c, b: (c * 8 + b, 0)),
        ],
        out_shape=[
            jax.ShapeDtypeStruct((_NPAD, _D), jnp.float32),
            jax.ShapeDtypeStruct((_NPAD, _D), jnp.float32),
            jax.ShapeDtypeStruct((2 * _NPAD, _D), jnp.float32),
        ],
    )
    loop_call = pl.kernel(
        _loop_body,
        out_type=(
            jax.ShapeDtypeStruct((2 * _NPAD, _D), jnp.float32),
            jax.ShapeDtypeStruct((2 * _NPAD, _D), jnp.float32),
            jax.ShapeDtypeStruct((2 * _NPAD, _D), jnp.float32),
        ),
        mesh=sc_mesh,
        scratch_types=(
            [pltpu.VMEM((_CPB, _CH), jnp.int32) for _ in range(4)]
            + [pltpu.VMEM((_RCH, _D), jnp.float32) for _ in range(2)]
            + [pltpu.VMEM_SHARED((_NPAD, _D), jnp.float32)]
            + [pltpu.SemaphoreType.DMA for _ in range(8)]
        ),
    )
    sums_call = pl.pallas_call(
        _sums_body,
        grid=(_N // _NBLK,),
        in_specs=[
            pl.BlockSpec((_NBLK, _D), lambda b: (b, 0)),
            pl.BlockSpec((_NBLK, _D), lambda b: (b, 0)),
            pl.BlockSpec((_D, _D), lambda b: (0, 0)),
            pl.BlockSpec((1, _D), lambda b: (0, 0)),
            pl.BlockSpec((_D, _D), lambda b: (0, 0)),
            pl.BlockSpec((1, _D), lambda b: (0, 0)),
            pl.BlockSpec((1, 1), lambda b: (0, 0)),
        ],
        out_specs=pl.BlockSpec((8, _D), lambda b: (0, 0)),
        out_shape=jax.ShapeDtypeStruct((8, _D), jnp.float32),
    )
    scores_call = pl.pallas_call(
        _scores_body,
        grid=(_N // _NBLK,),
        in_specs=[
            pl.BlockSpec((8, _D), lambda b: (0, 0)),
            pl.BlockSpec((_NBLK, _D), lambda b: (b, 0)),
            pl.BlockSpec((_NBLK, _D), lambda b: (b, 0)),
            pl.BlockSpec((_NBLK, _D), lambda b: (b, 0)),
            pl.BlockSpec((_NBLK, _D), lambda b: (b, 0)),
            pl.BlockSpec((_D, _D), lambda b: (0, 0)),
            pl.BlockSpec((1, _D), lambda b: (0, 0)),
            pl.BlockSpec((_D, _D), lambda b: (0, 0)),
            pl.BlockSpec((1, _D), lambda b: (0, 0)),
            pl.BlockSpec((_D, _D), lambda b: (0, 0)),
            pl.BlockSpec((1, 1), lambda b: (0, 0)),
            pl.BlockSpec((1, 1), lambda b: (0, 0)),
        ],
        out_specs=pl.BlockSpec((_NBLK, 4), lambda b: (b, 0)),
        out_shape=jax.ShapeDtypeStruct((_N, 4), jnp.float32),
    )
    return deg_call, prep_call, loop_call, sums_call, scores_call


def kernel(feat, shuf_feat, edge_index, W1, b1, Wlin, blin, Wbil, bbil,
           prelu_w):
    deg_call, prep_call, loop_call, sums_call, scores_call = _build_calls()

    src = edge_index[0].astype(jnp.int32)
    dst = edge_index[1].astype(jnp.int32)
    pad_ids = _N + (jnp.arange(_EPAD - _E, dtype=jnp.int32) % (_NPAD - _N))
    src1 = jnp.concatenate([src, pad_ids]).reshape(_EROWS, _CH)
    srcp = jnp.concatenate([src1, src1 + _NPAD], axis=0)
    dstp = jnp.concatenate([dst, pad_ids]).reshape(_EROWS, _CH)

    xf = jnp.pad(feat, ((0, _NPAD - _N), (0, 0)))
    xs = jnp.pad(shuf_feat, ((0, _NPAD - _N), (0, 0)))
    xcat = jnp.concatenate([xf, xs], axis=0)
    zer = jnp.zeros((_RCH, _D), jnp.float32)

    dego, degi = deg_call(srcp, dstp)
    degT = jnp.stack([dego, degi], axis=1)
    cn, nd, g0cat = prep_call(degT, xcat)
    _, hgcat, hkcat = loop_call(srcp, dstp, g0cat, xcat, cn, nd, zer)

    hgf = hgcat[:_N]
    hgs = hgcat[_NPAD:_NPAD + _N]
    hkf = hkcat[:_N]
    hks = hkcat[_NPAD:_NPAD + _N]

    b1r = b1.reshape(1, _D)
    blr = blin.reshape(1, _D)
    pwr = prelu_w.reshape(1, 1)
    bbr = bbil.reshape(1, 1)

    sums = sums_call(hgf, hkf, W1, b1r, Wlin, blr, pwr)
    scores = scores_call(sums, hgf, hkf, hgs, hks, W1, b1r, Wlin, blr,
                         Wbil, bbr, pwr)
    return scores.T.reshape(4 * _N)


# speedup vs baseline: 10.6983x; 1.0460x over previous
import functools

import jax
import jax.numpy as jnp
from jax import lax
from jax.experimental import pallas as pl
from jax.experimental.pallas import tpu as pltpu
from jax.experimental.pallas import tpu_sc as plsc

_N = 10000
_E = 320000
_D = 128
_K = 10
_ALPHA = 0.1

_NS = 16
_L = 16

_NPAD = 10240
_RPT = _NPAD // _NS
_RCH = 128
_NR = _RPT // _RCH
_CH = 128
_CPB = 16
_NBODY = 10
_EPT = _CPB * _CH * _NBODY
_EPAD = _NS * _EPT
_EROWS = _EPAD // _CH
_TROWS = _EPT // _CH


def _deg_body(src_ref, dst_ref, dego_ref, degi_ref, idxb, ones_v, stage_v,
              deg_sh, sdeg):
    cid = lax.axis_index("c")
    sid = lax.axis_index("s")

    def fill1(i, _):
        ones_v[pl.ds(i * _L, _L)] = jnp.ones((_L,), jnp.float32)
        return 0
    lax.fori_loop(0, _CH // _L, fill1, 0)

    def fill0(i, _):
        stage_v[pl.ds(i * _L, _L)] = jnp.zeros((_L,), jnp.float32)
        return 0
    lax.fori_loop(0, _RPT // _L, fill0, 0)
    pltpu.sync_copy(stage_v, deg_sh.at[pl.ds(sid * _RPT, _RPT)])
    plsc.subcore_barrier()

    def body(j, _):
        brow = sid * _TROWS + j * _CPB

        @pl.when(cid == 0)
        def _():
            pltpu.sync_copy(src_ref.at[pl.ds(brow, _CPB)], idxb)

        @pl.when(cid == 1)
        def _():
            pltpu.sync_copy(dst_ref.at[pl.ds(brow, _CPB)], idxb)

        dss = [pltpu.async_copy(ones_v, deg_sh.at[idxb.at[k]], sdeg,
                                add=True)
               for k in range(_CPB)]
        for d in dss:
            d.wait()
        return 0
    lax.fori_loop(0, _NBODY, body, 0)
    plsc.subcore_barrier()

    pltpu.sync_copy(deg_sh.at[pl.ds(sid * _RPT, _RPT)], stage_v)

    @pl.when(cid == 0)
    def _():
        pltpu.sync_copy(stage_v, dego_ref.at[pl.ds(sid * _RPT, _RPT)])

    @pl.when(cid == 1)
    def _():
        pltpu.sync_copy(stage_v, degi_ref.at[pl.ds(sid * _RPT, _RPT)])


_PBLK = _NPAD // 8


def _prep_body(degT_ref, x_ref, cn_ref, nd_ref, g0_ref, q0_ref):
    dg = degT_ref[...]
    ns = lax.rsqrt(jnp.maximum(dg[:, 0:1], 1.0))
    ndv = lax.rsqrt(jnp.maximum(dg[:, 1:2], 1.0))
    cn_ref[...] = jnp.broadcast_to((1.0 - _ALPHA) * ns * ndv, (_PBLK, _D))
    nd_ref[...] = jnp.broadcast_to(ndv, (_PBLK, _D))
    g0_ref[...] = ns * x_ref[...]
    q0_ref[...] = ((_ALPHA / (1.0 - _ALPHA))
                   * x_ref[...] * jnp.sqrt(jnp.maximum(dg[:, 1:2], 1.0)))


def _loop_body(src_ref, dst_ref, g0_ref, x_ref, cn_ref, nd_ref, zer_ref,
               q0_ref,
               gw_ref, hg_ref, hk_ref,
               is0, id0, is1, id1, bufp, bufq, agg_sh,
               si0, sd0, si1, sd1, sg0, sg1, ss0, ss1):
    cid = lax.axis_index("c")
    sid = lax.axis_index("s")
    row0 = sid * _RPT
    coff = cid * _NPAD
    sbase = cid * _EROWS + sid * _TROWS
    sgs = (sg0, sg1)
    sss = (ss0, ss1)
    bufs = (bufp, bufq)

    def init_chunk(rb, _):
        r = row0 + rb * _RCH
        pltpu.sync_copy(g0_ref.at[pl.ds(coff + r, _RCH)], bufp)
        pltpu.sync_copy(bufp, gw_ref.at[pl.ds(coff + r, _RCH)])
        pltpu.sync_copy(zer_ref, agg_sh.at[pl.ds(r, _RCH)])
        return 0
    lax.fori_loop(0, _NR, init_chunk, 0)
    plsc.subcore_barrier()

    ibufs = ((is0, id0, si0, sd0), (is1, id1, si1, sd1))

    def idx_start(jn, ib):
        isX, idX, siX, sdX = ib
        brow = jnp.minimum(sbase + jn * _CPB, 2 * _EROWS - _CPB)
        drow = jnp.minimum(sid * _TROWS + jn * _CPB, _EROWS - _CPB)
        pltpu.async_copy(src_ref.at[pl.ds(brow, _CPB)], isX, siX)
        pltpu.async_copy(dst_ref.at[pl.ds(drow, _CPB)], idX, sdX)

    def idx_wait(ib):
        isX, idX, siX, sdX = ib
        pltpu.make_async_copy(src_ref.at[pl.ds(0, _CPB)], isX, siX).wait()
        pltpu.make_async_copy(dst_ref.at[pl.ds(0, _CPB)], idX, sdX).wait()

    def edge_phase():
        idx_start(0, ibufs[0])

        def super_body(sb, _):
            for p in range(2):
                j = 2 * sb + p
                ibX = ibufs[p]
                idx_wait(ibX)
                idx_start(j + 1, ibufs[1 - p])
                isX, idX = ibX[0], ibX[1]
                dgs = [None] * _CPB
                dss = [None] * _CPB
                for k in range(_CPB):
                    if k >= 2:
                        dss[k - 2].wait()
                    dgs[k] = pltpu.async_copy(gw_ref.at[isX.at[k]],
                                              bufs[k % 2], sgs[k % 2])
                    if k >= 1:
                        dgs[k - 1].wait()
                        dss[k - 1] = pltpu.async_copy(
                            bufs[(k - 1) % 2], agg_sh.at[idX.at[k - 1]],
                            sss[(k - 1) % 2], add=True)
                km = _CPB - 1
                dgs[km].wait()
                dss[km] = pltpu.async_copy(
                    bufs[km % 2], agg_sh.at[idX.at[km]], sss[km % 2],
                    add=True)
                dss[km - 1].wait()
                dss[km].wait()
            return 0
        lax.fori_loop(0, _NBODY // 2, super_body, 0)
        idx_wait(ibufs[0])

    def ew(rows, fn):
        @plsc.parallel_loop(0, rows, unroll=4)
        def _(rr):
            for cc in range(_D // _L):
                fn(rr, pl.ds(cc * _L, _L))

    def dense_mid(refill_q0):
        def chunk(rb, _):
            r = row0 + rb * _RCH
            rg = coff + r
            pltpu.sync_copy(agg_sh.at[pl.ds(r, _RCH)], bufp)
            pltpu.sync_copy(cn_ref.at[pl.ds(r, _RCH)], bufq)

            def mul(rr, sl):
                bufp[rr, sl] = bufp[rr, sl] * bufq[rr, sl]
            ew(_RCH, mul)
            pltpu.sync_copy(bufp, gw_ref.at[pl.ds(rg, _RCH)])
            if refill_q0:
                pltpu.sync_copy(q0_ref.at[pl.ds(rg, _RCH)],
                                agg_sh.at[pl.ds(r, _RCH)])
            else:
                pltpu.sync_copy(zer_ref, agg_sh.at[pl.ds(r, _RCH)])
            return 0
        return chunk

    def dense_t0(rb, _):
        h = _RCH // 2
        r = row0 + rb * h
        rg = coff + r
        pltpu.sync_copy(agg_sh.at[pl.ds(r, h)], bufp.at[pl.ds(0, h)])
        pltpu.sync_copy(nd_ref.at[pl.ds(r, h)], bufq.at[pl.ds(0, h)])

        def hgmul(rr, sl):
            bufq[rr, sl] = bufq[rr, sl] * bufp[rr, sl]
        ew(h, hgmul)
        pltpu.sync_copy(bufq.at[pl.ds(0, h)], hg_ref.at[pl.ds(rg, h)])
        pltpu.sync_copy(cn_ref.at[pl.ds(r, h)], bufq.at[pl.ds(0, h)])

        def mul(rr, sl):
            bufp[rr, sl] = bufp[rr, sl] * bufq[rr, sl]
        ew(h, mul)
        pltpu.sync_copy(g0_ref.at[pl.ds(rg, h)], bufq.at[pl.ds(0, h)])

        def axpy(rr, sl):
            bufp[rr, sl] = bufp[rr, sl] + _ALPHA * bufq[rr, sl]
        ew(h, axpy)
        pltpu.sync_copy(bufp.at[pl.ds(0, h)], gw_ref.at[pl.ds(rg, h)])
        pltpu.sync_copy(q0_ref.at[pl.ds(rg, h)], agg_sh.at[pl.ds(r, h)])
        return 0

    def dense_t9(rb, _):
        r = row0 + rb * _RCH
        rg = coff + r
        pltpu.sync_copy(agg_sh.at[pl.ds(r, _RCH)], bufp)
        pltpu.sync_copy(nd_ref.at[pl.ds(r, _RCH)], bufq)

        def mul9(rr, sl):
            bufp[rr, sl] = (1.0 - _ALPHA) * bufp[rr, sl] * bufq[rr, sl]
        ew(_RCH, mul9)
        pltpu.sync_copy(x_ref.at[pl.ds(rg, _RCH)], bufq)

        def axpy(rr, sl):
            bufp[rr, sl] = bufp[rr, sl] + _ALPHA * bufq[rr, sl]
        ew(_RCH, axpy)
        pltpu.sync_copy(bufp, hk_ref.at[pl.ds(rg, _RCH)])
        return 0

    for t in range(_K):
        edge_phase()
        plsc.subcore_barrier()
        if t == 0:
            lax.fori_loop(0, 2 * _NR, dense_t0, 0)
        elif t == _K - 1:
            lax.fori_loop(0, _NR, dense_t9, 0)
        else:
            lax.fori_loop(0, _NR, dense_mid(t < _K - 2), 0)
        plsc.subcore_barrier()


_NBLK = 2000


def _prelu(x, w):
    return jnp.where(x > 0, x, w * x)


def _sums_body(hgf_ref, hkf_ref, w1_ref, b1_ref, wl_ref, bl_ref, pw_ref,
               sums_ref):
    i = pl.program_id(0)
    w = pw_ref[0, 0]
    h1 = _prelu(jnp.dot(hgf_ref[...], w1_ref[...],
                        preferred_element_type=jnp.float32) + b1_ref[...], w)
    h2 = _prelu(jnp.dot(hkf_ref[...], wl_ref[...],
                        preferred_element_type=jnp.float32) + bl_ref[...], w)

    @pl.when(i == 0)
    def _():
        sums_ref[...] = jnp.zeros((8, _D), jnp.float32)

    sums_ref[0:1, :] = sums_ref[0:1, :] + jnp.sum(h1, axis=0, keepdims=True)
    sums_ref[1:2, :] = sums_ref[1:2, :] + jnp.sum(h2, axis=0, keepdims=True)


def _scores_body(sums_ref, hgf_ref, hkf_ref, hgs_ref, hks_ref,
                 w1_ref, b1_ref, wl_ref, bl_ref, wb_ref, bb_ref, pw_ref,
                 out_ref):
    w = pw_ref[0, 0]
    bb = bb_ref[0, 0]
    c1 = jax.nn.sigmoid(sums_ref[0:1, :] * (1.0 / _N))
    c2 = jax.nn.sigmoid(sums_ref[1:2, :] * (1.0 / _N))
    q1 = lax.dot_general(wb_ref[...], c1, (((1,), (1,)), ((), ())),
                         preferred_element_type=jnp.float32)
    q2 = lax.dot_general(wb_ref[...], c2, (((1,), (1,)), ((), ())),
                         preferred_element_type=jnp.float32)
    h1 = _prelu(jnp.dot(hgf_ref[...], w1_ref[...],
                        preferred_element_type=jnp.float32) + b1_ref[...], w)
    h2 = _prelu(jnp.dot(hkf_ref[...], wl_ref[...],
                        preferred_element_type=jnp.float32) + bl_ref[...], w)
    h3 = _prelu(jnp.dot(hgs_ref[...], w1_ref[...],
                        preferred_element_type=jnp.float32) + b1_ref[...], w)
    h4 = _prelu(jnp.dot(hks_ref[...], wl_ref[...],
                        preferred_element_type=jnp.float32) + bl_ref[...], w)
    out_ref[:, 0:1] = jnp.dot(h2, q1, preferred_element_type=jnp.float32) + bb
    out_ref[:, 1:2] = jnp.dot(h1, q2, preferred_element_type=jnp.float32) + bb
    out_ref[:, 2:3] = jnp.dot(h4, q1, preferred_element_type=jnp.float32) + bb
    out_ref[:, 3:4] = jnp.dot(h3, q2, preferred_element_type=jnp.float32) + bb


@functools.cache
def _build_calls():
    sc_mesh = plsc.VectorSubcoreMesh(core_axis_name="c", subcore_axis_name="s")
    deg_call = pl.kernel(
        _deg_body,
        out_type=(
            jax.ShapeDtypeStruct((_NPAD,), jnp.float32),
            jax.ShapeDtypeStruct((_NPAD,), jnp.float32),
        ),
        mesh=sc_mesh,
        scratch_types=[
            pltpu.VMEM((_CPB, _CH), jnp.int32),
            pltpu.VMEM((_CH,), jnp.float32),
            pltpu.VMEM((_RPT,), jnp.float32),
            pltpu.VMEM_SHARED((_NPAD,), jnp.float32),
            pltpu.SemaphoreType.DMA,
        ],
    )
    prep_call = pl.pallas_call(
        _prep_body,
        grid=(2, 8),
        in_specs=[
            pl.BlockSpec((_PBLK, 2), lambda c, b: (b, 0)),
            pl.BlockSpec((_PBLK, _D), lambda c, b: (c * 8 + b, 0)),
        ],
        out_specs=[
            pl.BlockSpec((_PBLK, _D), lambda c, b: (b, 0)),
            pl.BlockSpec((_PBLK, _D), lambda c, b: (b, 0)),
            pl.BlockSpec((_PBLK, _D), lambda c, b: (c * 8 + b, 0)),
            pl.BlockSpec((_PBLK, _D), lambda c, b: (c * 8 + b, 0)),
        ],
        out_shape=[
            jax.ShapeDtypeStruct((_NPAD, _D), jnp.float32),
            jax.ShapeDtypeStruct((_NPAD, _D), jnp.float32),
            jax.ShapeDtypeStruct((2 * _NPAD, _D), jnp.float32),
            jax.ShapeDtypeStruct((2 * _NPAD, _D), jnp.float32),
        ],
    )
    loop_call = pl.kernel(
        _loop_body,
        out_type=(
            jax.ShapeDtypeStruct((2 * _NPAD, _D), jnp.float32),
            jax.ShapeDtypeStruct((2 * _NPAD, _D), jnp.float32),
            jax.ShapeDtypeStruct((2 * _NPAD, _D), jnp.float32),
        ),
        mesh=sc_mesh,
        scratch_types=(
            [pltpu.VMEM((_CPB, _CH), jnp.int32) for _ in range(4)]
            + [pltpu.VMEM((_RCH, _D), jnp.float32) for _ in range(2)]
            + [pltpu.VMEM_SHARED((_NPAD, _D), jnp.float32)]
            + [pltpu.SemaphoreType.DMA for _ in range(8)]
        ),
    )
    sums_call = pl.pallas_call(
        _sums_body,
        grid=(_N // _NBLK,),
        in_specs=[
            pl.BlockSpec((_NBLK, _D), lambda b: (b, 0)),
            pl.BlockSpec((_NBLK, _D), lambda b: (b, 0)),
            pl.BlockSpec((_D, _D), lambda b: (0, 0)),
            pl.BlockSpec((1, _D), lambda b: (0, 0)),
            pl.BlockSpec((_D, _D), lambda b: (0, 0)),
            pl.BlockSpec((1, _D), lambda b: (0, 0)),
            pl.BlockSpec((1, 1), lambda b: (0, 0)),
        ],
        out_specs=pl.BlockSpec((8, _D), lambda b: (0, 0)),
        out_shape=jax.ShapeDtypeStruct((8, _D), jnp.float32),
    )
    scores_call = pl.pallas_call(
        _scores_body,
        grid=(_N // _NBLK,),
        in_specs=[
            pl.BlockSpec((8, _D), lambda b: (0, 0)),
            pl.BlockSpec((_NBLK, _D), lambda b: (b, 0)),
            pl.BlockSpec((_NBLK, _D), lambda b: (b, 0)),
            pl.BlockSpec((_NBLK, _D), lambda b: (b, 0)),
            pl.BlockSpec((_NBLK, _D), lambda b: (b, 0)),
            pl.BlockSpec((_D, _D), lambda b: (0, 0)),
            pl.BlockSpec((1, _D), lambda b: (0, 0)),
            pl.BlockSpec((_D, _D), lambda b: (0, 0)),
            pl.BlockSpec((1, _D), lambda b: (0, 0)),
            pl.BlockSpec((_D, _D), lambda b: (0, 0)),
            pl.BlockSpec((1, 1), lambda b: (0, 0)),
            pl.BlockSpec((1, 1), lambda b: (0, 0)),
        ],
        out_specs=pl.BlockSpec((_NBLK, 4), lambda b: (b, 0)),
        out_shape=jax.ShapeDtypeStruct((_N, 4), jnp.float32),
    )
    return deg_call, prep_call, loop_call, sums_call, scores_call


def kernel(feat, shuf_feat, edge_index, W1, b1, Wlin, blin, Wbil, bbil,
           prelu_w):
    deg_call, prep_call, loop_call, sums_call, scores_call = _build_calls()

    src = edge_index[0].astype(jnp.int32)
    dst = edge_index[1].astype(jnp.int32)
    pad_ids = _N + (jnp.arange(_EPAD - _E, dtype=jnp.int32) % (_NPAD - _N))
    src1 = jnp.concatenate([src, pad_ids]).reshape(_EROWS, _CH)
    srcp = jnp.concatenate([src1, src1 + _NPAD], axis=0)
    dstp = jnp.concatenate([dst, pad_ids]).reshape(_EROWS, _CH)

    xf = jnp.pad(feat, ((0, _NPAD - _N), (0, 0)))
    xs = jnp.pad(shuf_feat, ((0, _NPAD - _N), (0, 0)))
    xcat = jnp.concatenate([xf, xs], axis=0)
    zer = jnp.zeros((_RCH, _D), jnp.float32)

    dego, degi = deg_call(srcp, dstp)
    degT = jnp.stack([dego, degi], axis=1)
    cn, nd, g0cat, q0cat = prep_call(degT, xcat)
    _, hgcat, hkcat = loop_call(srcp, dstp, g0cat, xcat, cn, nd, zer, q0cat)

    hgf = hgcat[:_N]
    hgs = hgcat[_NPAD:_NPAD + _N]
    hkf = hkcat[:_N]
    hks = hkcat[_NPAD:_NPAD + _N]

    b1r = b1.reshape(1, _D)
    blr = blin.reshape(1, _D)
    pwr = prelu_w.reshape(1, 1)
    bbr = bbil.reshape(1, 1)

    sums = sums_call(hgf, hkf, W1, b1r, Wlin, blr, pwr)
    scores = scores_call(sums, hgf, hkf, hgs, hks, W1, b1r, Wlin, blr,
                         Wbil, bbr, pwr)
    return scores.T.reshape(4 * _N)


# speedup vs baseline: 10.9404x; 1.0226x over previous
import functools

import jax
import jax.numpy as jnp
from jax import lax
from jax.experimental import pallas as pl
from jax.experimental.pallas import tpu as pltpu
from jax.experimental.pallas import tpu_sc as plsc

_N = 10000
_E = 320000
_D = 128
_K = 10
_ALPHA = 0.1

_NS = 16
_L = 16

_NPAD = 10240
_RPT = _NPAD // _NS
_RCH = 128
_NR = _RPT // _RCH
_CH = 128
_CPB = 16
_NBODY = 10
_EPT = _CPB * _CH * _NBODY
_EPAD = _NS * _EPT
_EROWS = _EPAD // _CH
_TROWS = _EPT // _CH


def _deg_body(src_ref, dst_ref, dego_ref, degi_ref, idxb, ones_v, stage_v,
              deg_sh, sdeg):
    cid = lax.axis_index("c")
    sid = lax.axis_index("s")

    def fill1(i, _):
        ones_v[pl.ds(i * _L, _L)] = jnp.ones((_L,), jnp.float32)
        return 0
    lax.fori_loop(0, _CH // _L, fill1, 0)

    def fill0(i, _):
        stage_v[pl.ds(i * _L, _L)] = jnp.zeros((_L,), jnp.float32)
        return 0
    lax.fori_loop(0, _RPT // _L, fill0, 0)
    pltpu.sync_copy(stage_v, deg_sh.at[pl.ds(sid * _RPT, _RPT)])
    plsc.subcore_barrier()

    def body(j, _):
        brow = sid * _TROWS + j * _CPB

        @pl.when(cid == 0)
        def _():
            pltpu.sync_copy(src_ref.at[pl.ds(brow, _CPB)], idxb)

        @pl.when(cid == 1)
        def _():
            pltpu.sync_copy(dst_ref.at[pl.ds(brow, _CPB)], idxb)

        dss = [pltpu.async_copy(ones_v, deg_sh.at[idxb.at[k]], sdeg,
                                add=True)
               for k in range(_CPB)]
        for d in dss:
            d.wait()
        return 0
    lax.fori_loop(0, _NBODY, body, 0)
    plsc.subcore_barrier()

    pltpu.sync_copy(deg_sh.at[pl.ds(sid * _RPT, _RPT)], stage_v)

    @pl.when(cid == 0)
    def _():
        pltpu.sync_copy(stage_v, dego_ref.at[pl.ds(sid * _RPT, _RPT)])

    @pl.when(cid == 1)
    def _():
        pltpu.sync_copy(stage_v, degi_ref.at[pl.ds(sid * _RPT, _RPT)])


_PBLK = _NPAD // 8


def _prep_body(degT_ref, x_ref, cn_ref, nd_ref, g0_ref, q0_ref):
    dg = degT_ref[...]
    ns = lax.rsqrt(jnp.maximum(dg[:, 0:1], 1.0))
    ndv = lax.rsqrt(jnp.maximum(dg[:, 1:2], 1.0))
    cn_ref[...] = jnp.broadcast_to((1.0 - _ALPHA) * ns * ndv, (_PBLK, _D))
    nd_ref[...] = jnp.broadcast_to(ndv, (_PBLK, _D))
    g0_ref[...] = ns * x_ref[...]
    q0_ref[...] = ((_ALPHA / (1.0 - _ALPHA))
                   * x_ref[...] * jnp.sqrt(jnp.maximum(dg[:, 1:2], 1.0)))


def _loop_body(src_ref, dst_ref, g0_ref, x_ref, cn_ref, nd_ref, zer_ref,
               q0_ref,
               gw_ref, hg_ref, hk_ref,
               is0, id0, is1, id1, bufp, bufq, agg_sh,
               si0, sd0, si1, sd1, sg0, sg1, ss0, ss1):
    cid = lax.axis_index("c")
    sid = lax.axis_index("s")
    row0 = sid * _RPT
    coff = cid * _NPAD
    sbase = cid * _EROWS + sid * _TROWS
    sgs = (sg0, sg1)
    sss = (ss0, ss1)
    bufs = (bufp, bufq)

    def init_chunk(rb, _):
        r = row0 + rb * _RCH
        pltpu.sync_copy(g0_ref.at[pl.ds(coff + r, _RCH)], bufp)
        pltpu.sync_copy(bufp, gw_ref.at[pl.ds(coff + r, _RCH)])
        pltpu.sync_copy(zer_ref, agg_sh.at[pl.ds(r, _RCH)])
        return 0
    lax.fori_loop(0, _NR, init_chunk, 0)
    plsc.subcore_barrier()

    ibufs = ((is0, id0, si0, sd0), (is1, id1, si1, sd1))

    def idx_start(jn, ib):
        isX, idX, siX, sdX = ib
        brow = jnp.minimum(sbase + jn * _CPB, 2 * _EROWS - _CPB)
        drow = jnp.minimum(sid * _TROWS + jn * _CPB, _EROWS - _CPB)
        pltpu.async_copy(src_ref.at[pl.ds(brow, _CPB)], isX, siX)
        pltpu.async_copy(dst_ref.at[pl.ds(drow, _CPB)], idX, sdX)

    def idx_wait(ib):
        isX, idX, siX, sdX = ib
        pltpu.make_async_copy(src_ref.at[pl.ds(0, _CPB)], isX, siX).wait()
        pltpu.make_async_copy(dst_ref.at[pl.ds(0, _CPB)], idX, sdX).wait()

    def edge_phase():
        idx_start(0, ibufs[0])

        def super_body(sb, _):
            for p in range(2):
                j = 2 * sb + p
                ibX = ibufs[p]
                idx_wait(ibX)
                idx_start(j + 1, ibufs[1 - p])
                isX, idX = ibX[0], ibX[1]
                dgs = [None] * _CPB
                dss = [None] * _CPB
                for k in range(_CPB):
                    if k >= 2:
                        dss[k - 2].wait()
                    dgs[k] = pltpu.async_copy(gw_ref.at[isX.at[k]],
                                              bufs[k % 2], sgs[k % 2])
                    if k >= 1:
                        dgs[k - 1].wait()
                        dss[k - 1] = pltpu.async_copy(
                            bufs[(k - 1) % 2], agg_sh.at[idX.at[k - 1]],
                            sss[(k - 1) % 2], add=True)
                km = _CPB - 1
                dgs[km].wait()
                dss[km] = pltpu.async_copy(
                    bufs[km % 2], agg_sh.at[idX.at[km]], sss[km % 2],
                    add=True)
                dss[km - 1].wait()
                dss[km].wait()
            return 0
        lax.fori_loop(0, _NBODY // 2, super_body, 0)
        idx_wait(ibufs[0])

    def ew(rows, fn):
        @plsc.parallel_loop(0, rows, unroll=4)
        def _(rr):
            for cc in range(_D // _L):
                fn(rr, pl.ds(cc * _L, _L))

    def dense_mid(refill_q0):
        def chunk(rb, _):
            r = row0 + rb * _RCH
            rg = coff + r
            da = pltpu.async_copy(agg_sh.at[pl.ds(r, _RCH)], bufp, sg0)
            dc = pltpu.async_copy(cn_ref.at[pl.ds(r, _RCH)], bufq, sg1)
            da.wait()
            dc.wait()

            def mul(rr, sl):
                bufp[rr, sl] = bufp[rr, sl] * bufq[rr, sl]
            ew(_RCH, mul)
            dw = pltpu.async_copy(bufp, gw_ref.at[pl.ds(rg, _RCH)], ss0)
            if refill_q0:
                dz = pltpu.async_copy(q0_ref.at[pl.ds(rg, _RCH)],
                                      agg_sh.at[pl.ds(r, _RCH)], ss1)
            else:
                dz = pltpu.async_copy(zer_ref, agg_sh.at[pl.ds(r, _RCH)],
                                      ss1)
            dw.wait()
            dz.wait()
            return 0
        return chunk

    def dense_t0(rb, _):
        h = _RCH // 2
        r = row0 + rb * h
        rg = coff + r
        pltpu.sync_copy(agg_sh.at[pl.ds(r, h)], bufp.at[pl.ds(0, h)])
        pltpu.sync_copy(nd_ref.at[pl.ds(r, h)], bufq.at[pl.ds(0, h)])

        def hgmul(rr, sl):
            bufq[rr, sl] = bufq[rr, sl] * bufp[rr, sl]
        ew(h, hgmul)
        pltpu.sync_copy(bufq.at[pl.ds(0, h)], hg_ref.at[pl.ds(rg, h)])
        pltpu.sync_copy(cn_ref.at[pl.ds(r, h)], bufq.at[pl.ds(0, h)])

        def mul(rr, sl):
            bufp[rr, sl] = bufp[rr, sl] * bufq[rr, sl]
        ew(h, mul)
        pltpu.sync_copy(g0_ref.at[pl.ds(rg, h)], bufq.at[pl.ds(0, h)])

        def axpy(rr, sl):
            bufp[rr, sl] = bufp[rr, sl] + _ALPHA * bufq[rr, sl]
        ew(h, axpy)
        pltpu.sync_copy(bufp.at[pl.ds(0, h)], gw_ref.at[pl.ds(rg, h)])
        pltpu.sync_copy(q0_ref.at[pl.ds(rg, h)], agg_sh.at[pl.ds(r, h)])
        return 0

    def dense_t9(rb, _):
        r = row0 + rb * _RCH
        rg = coff + r
        pltpu.sync_copy(agg_sh.at[pl.ds(r, _RCH)], bufp)
        pltpu.sync_copy(nd_ref.at[pl.ds(r, _RCH)], bufq)

        def mul9(rr, sl):
            bufp[rr, sl] = (1.0 - _ALPHA) * bufp[rr, sl] * bufq[rr, sl]
        ew(_RCH, mul9)
        pltpu.sync_copy(x_ref.at[pl.ds(rg, _RCH)], bufq)

        def axpy(rr, sl):
            bufp[rr, sl] = bufp[rr, sl] + _ALPHA * bufq[rr, sl]
        ew(_RCH, axpy)
        pltpu.sync_copy(bufp, hk_ref.at[pl.ds(rg, _RCH)])
        return 0

    for t in range(_K):
        edge_phase()
        plsc.subcore_barrier()
        if t == 0:
            lax.fori_loop(0, 2 * _NR, dense_t0, 0)
        elif t == _K - 1:
            lax.fori_loop(0, _NR, dense_t9, 0)
        else:
            lax.fori_loop(0, _NR, dense_mid(t < _K - 2), 0)
        plsc.subcore_barrier()


_NBLK = 2000


def _prelu(x, w):
    return jnp.where(x > 0, x, w * x)


def _sums_body(hgf_ref, hkf_ref, w1_ref, b1_ref, wl_ref, bl_ref, pw_ref,
               sums_ref):
    i = pl.program_id(0)
    w = pw_ref[0, 0]
    h1 = _prelu(jnp.dot(hgf_ref[...], w1_ref[...],
                        preferred_element_type=jnp.float32) + b1_ref[...], w)
    h2 = _prelu(jnp.dot(hkf_ref[...], wl_ref[...],
                        preferred_element_type=jnp.float32) + bl_ref[...], w)

    @pl.when(i == 0)
    def _():
        sums_ref[...] = jnp.zeros((8, _D), jnp.float32)

    sums_ref[0:1, :] = sums_ref[0:1, :] + jnp.sum(h1, axis=0, keepdims=True)
    sums_ref[1:2, :] = sums_ref[1:2, :] + jnp.sum(h2, axis=0, keepdims=True)


def _scores_body(sums_ref, hgf_ref, hkf_ref, hgs_ref, hks_ref,
                 w1_ref, b1_ref, wl_ref, bl_ref, wb_ref, bb_ref, pw_ref,
                 out_ref):
    w = pw_ref[0, 0]
    bb = bb_ref[0, 0]
    c1 = jax.nn.sigmoid(sums_ref[0:1, :] * (1.0 / _N))
    c2 = jax.nn.sigmoid(sums_ref[1:2, :] * (1.0 / _N))
    q1 = lax.dot_general(wb_ref[...], c1, (((1,), (1,)), ((), ())),
                         preferred_element_type=jnp.float32)
    q2 = lax.dot_general(wb_ref[...], c2, (((1,), (1,)), ((), ())),
                         preferred_element_type=jnp.float32)
    h1 = _prelu(jnp.dot(hgf_ref[...], w1_ref[...],
                        preferred_element_type=jnp.float32) + b1_ref[...], w)
    h2 = _prelu(jnp.dot(hkf_ref[...], wl_ref[...],
                        preferred_element_type=jnp.float32) + bl_ref[...], w)
    h3 = _prelu(jnp.dot(hgs_ref[...], w1_ref[...],
                        preferred_element_type=jnp.float32) + b1_ref[...], w)
    h4 = _prelu(jnp.dot(hks_ref[...], wl_ref[...],
                        preferred_element_type=jnp.float32) + bl_ref[...], w)
    out_ref[:, 0:1] = jnp.dot(h2, q1, preferred_element_type=jnp.float32) + bb
    out_ref[:, 1:2] = jnp.dot(h1, q2, preferred_element_type=jnp.float32) + bb
    out_ref[:, 2:3] = jnp.dot(h4, q1, preferred_element_type=jnp.float32) + bb
    out_ref[:, 3:4] = jnp.dot(h3, q2, preferred_element_type=jnp.float32) + bb


@functools.cache
def _build_calls():
    sc_mesh = plsc.VectorSubcoreMesh(core_axis_name="c", subcore_axis_name="s")
    deg_call = pl.kernel(
        _deg_body,
        out_type=(
            jax.ShapeDtypeStruct((_NPAD,), jnp.float32),
            jax.ShapeDtypeStruct((_NPAD,), jnp.float32),
        ),
        mesh=sc_mesh,
        scratch_types=[
            pltpu.VMEM((_CPB, _CH), jnp.int32),
            pltpu.VMEM((_CH,), jnp.float32),
            pltpu.VMEM((_RPT,), jnp.float32),
            pltpu.VMEM_SHARED((_NPAD,), jnp.float32),
            pltpu.SemaphoreType.DMA,
        ],
    )
    prep_call = pl.pallas_call(
        _prep_body,
        grid=(2, 8),
        in_specs=[
            pl.BlockSpec((_PBLK, 2), lambda c, b: (b, 0)),
            pl.BlockSpec((_PBLK, _D), lambda c, b: (c * 8 + b, 0)),
        ],
        out_specs=[
            pl.BlockSpec((_PBLK, _D), lambda c, b: (b, 0)),
            pl.BlockSpec((_PBLK, _D), lambda c, b: (b, 0)),
            pl.BlockSpec((_PBLK, _D), lambda c, b: (c * 8 + b, 0)),
            pl.BlockSpec((_PBLK, _D), lambda c, b: (c * 8 + b, 0)),
        ],
        out_shape=[
            jax.ShapeDtypeStruct((_NPAD, _D), jnp.float32),
            jax.ShapeDtypeStruct((_NPAD, _D), jnp.float32),
            jax.ShapeDtypeStruct((2 * _NPAD, _D), jnp.float32),
            jax.ShapeDtypeStruct((2 * _NPAD, _D), jnp.float32),
        ],
    )
    loop_call = pl.kernel(
        _loop_body,
        out_type=(
            jax.ShapeDtypeStruct((2 * _NPAD, _D), jnp.float32),
            jax.ShapeDtypeStruct((2 * _NPAD, _D), jnp.float32),
            jax.ShapeDtypeStruct((2 * _NPAD, _D), jnp.float32),
        ),
        mesh=sc_mesh,
        scratch_types=(
            [pltpu.VMEM((_CPB, _CH), jnp.int32) for _ in range(4)]
            + [pltpu.VMEM((_RCH, _D), jnp.float32) for _ in range(2)]
            + [pltpu.VMEM_SHARED((_NPAD, _D), jnp.float32)]
            + [pltpu.SemaphoreType.DMA for _ in range(8)]
        ),
    )
    sums_call = pl.pallas_call(
        _sums_body,
        grid=(_N // _NBLK,),
        in_specs=[
            pl.BlockSpec((_NBLK, _D), lambda b: (b, 0)),
            pl.BlockSpec((_NBLK, _D), lambda b: (b, 0)),
            pl.BlockSpec((_D, _D), lambda b: (0, 0)),
            pl.BlockSpec((1, _D), lambda b: (0, 0)),
            pl.BlockSpec((_D, _D), lambda b: (0, 0)),
            pl.BlockSpec((1, _D), lambda b: (0, 0)),
            pl.BlockSpec((1, 1), lambda b: (0, 0)),
        ],
        out_specs=pl.BlockSpec((8, _D), lambda b: (0, 0)),
        out_shape=jax.ShapeDtypeStruct((8, _D), jnp.float32),
    )
    scores_call = pl.pallas_call(
        _scores_body,
        grid=(_N // _NBLK,),
        in_specs=[
            pl.BlockSpec((8, _D), lambda b: (0, 0)),
            pl.BlockSpec((_NBLK, _D), lambda b: (b, 0)),
            pl.BlockSpec((_NBLK, _D), lambda b: (b, 0)),
            pl.BlockSpec((_NBLK, _D), lambda b: (b, 0)),
            pl.BlockSpec((_NBLK, _D), lambda b: (b, 0)),
            pl.BlockSpec((_D, _D), lambda b: (0, 0)),
            pl.BlockSpec((1, _D), lambda b: (0, 0)),
            pl.BlockSpec((_D, _D), lambda b: (0, 0)),
            pl.BlockSpec((1, _D), lambda b: (0, 0)),
            pl.BlockSpec((_D, _D), lambda b: (0, 0)),
            pl.BlockSpec((1, 1), lambda b: (0, 0)),
            pl.BlockSpec((1, 1), lambda b: (0, 0)),
        ],
        out_specs=pl.BlockSpec((_NBLK, 4), lambda b: (b, 0)),
        out_shape=jax.ShapeDtypeStruct((_N, 4), jnp.float32),
    )
    return deg_call, prep_call, loop_call, sums_call, scores_call


def kernel(feat, shuf_feat, edge_index, W1, b1, Wlin, blin, Wbil, bbil,
           prelu_w):
    deg_call, prep_call, loop_call, sums_call, scores_call = _build_calls()

    src = edge_index[0].astype(jnp.int32)
    dst = edge_index[1].astype(jnp.int32)
    pad_ids = _N + (jnp.arange(_EPAD - _E, dtype=jnp.int32) % (_NPAD - _N))
    src1 = jnp.concatenate([src, pad_ids]).reshape(_EROWS, _CH)
    srcp = jnp.concatenate([src1, src1 + _NPAD], axis=0)
    dstp = jnp.concatenate([dst, pad_ids]).reshape(_EROWS, _CH)

    xf = jnp.pad(feat, ((0, _NPAD - _N), (0, 0)))
    xs = jnp.pad(shuf_feat, ((0, _NPAD - _N), (0, 0)))
    xcat = jnp.concatenate([xf, xs], axis=0)
    zer = jnp.zeros((_RCH, _D), jnp.float32)

    dego, degi = deg_call(srcp, dstp)
    degT = jnp.stack([dego, degi], axis=1)
    cn, nd, g0cat, q0cat = prep_call(degT, xcat)
    _, hgcat, hkcat = loop_call(srcp, dstp, g0cat, xcat, cn, nd, zer, q0cat)

    hgf = hgcat[:_N]
    hgs = hgcat[_NPAD:_NPAD + _N]
    hkf = hkcat[:_N]
    hks = hkcat[_NPAD:_NPAD + _N]

    b1r = b1.reshape(1, _D)
    blr = blin.reshape(1, _D)
    pwr = prelu_w.reshape(1, 1)
    bbr = bbil.reshape(1, 1)

    sums = sums_call(hgf, hkf, W1, b1r, Wlin, blr, pwr)
    scores = scores_call(sums, hgf, hkf, hgs, hks, W1, b1r, Wlin, blr,
                         Wbil, bbr, pwr)
    return scores.T.reshape(4 * _N)
